# Initial kernel scaffold; baseline (speedup 1.0000x reference)
import functools

import jax
import jax.numpy as jnp
from jax import lax
from jax.experimental import pallas as pl
from jax.experimental.pallas import tpu as pltpu
from jax.experimental.pallas import tpu_sc as plsc

N_NODES = 10000
E_EDGES = 320000
IN_FEATS = 128
H_FEATS = 128
OUT_FEATS = 64

N_CORES = 2
N_SUBCORES = 16
NW = N_CORES * N_SUBCORES
SUB = 128
NSUB = 4
BLK = SUB * NSUB
BLOCKS_PER_TILE = 20
E_PAD = NW * BLOCKS_PER_TILE * BLK
N_ACC = 10016
ROWS_OUT = N_NODES // N_SUBCORES
ROWS_ACC = N_ACC // N_SUBCORES
BN = 1000

_MESH = plsc.VectorSubcoreMesh(core_axis_name="c", subcore_axis_name="s")


def _degree_body(dst2d, ones_hbm, zeros_hbm, out_hbm, idx_v, ones_v, acc):
  c = lax.axis_index("c")
  s = lax.axis_index("s")
  wid = c * N_SUBCORES + s
  ra = pl.ds(s * ROWS_ACC, ROWS_ACC)
  pltpu.sync_copy(zeros_hbm.at[ra], acc.at[ra])
  pltpu.sync_copy(ones_hbm, ones_v)
  plsc.subcore_barrier()
  base = wid * BLOCKS_PER_TILE

  @pl.loop(0, BLOCKS_PER_TILE)
  def _(b):
    row0 = (base + b) * NSUB
    pltpu.sync_copy(dst2d.at[pl.ds(row0, NSUB)], idx_v)
    for j in range(NSUB):
      pltpu.sync_copy(ones_v, acc.at[idx_v.at[j]], add=True)

  plsc.subcore_barrier()
  ro = pl.ds(s * ROWS_OUT, ROWS_OUT)
  pltpu.sync_copy(acc.at[ro], out_hbm.at[c, ro])


_degree = pl.kernel(
    _degree_body,
    out_type=jax.ShapeDtypeStruct((N_CORES, N_NODES, 16), jnp.float32),
    mesh=_MESH,
    scratch_types=[
        pltpu.VMEM((NSUB, SUB), jnp.int32),
        pltpu.VMEM((SUB, 16), jnp.float32),
        pltpu.VMEM_SHARED((N_ACC, 16), jnp.float32),
    ],
)


def _make_conv(feats):
  def body(src2d, dst2d, table, zeros_hbm, out_hbm, src_v, dst_v, rows_v, acc,
           sem):
    c = lax.axis_index("c")
    s = lax.axis_index("s")
    wid = c * N_SUBCORES + s
    ra = pl.ds(s * ROWS_ACC, ROWS_ACC)

    @pl.when(c == 0)
    def _():
      pltpu.sync_copy(table.at[ra], acc.at[ra])

    @pl.when(c != 0)
    def _():
      pltpu.sync_copy(zeros_hbm.at[ra], acc.at[ra])

    plsc.subcore_barrier()
    base = wid * BLOCKS_PER_TILE

    @pl.loop(0, BLOCKS_PER_TILE)
    def _(b):
      row0 = (base + b) * NSUB
      pltpu.sync_copy(src2d.at[pl.ds(row0, NSUB)], src_v)
      pltpu.sync_copy(dst2d.at[pl.ds(row0, NSUB)], dst_v)
      descs = [
          pltpu.async_copy(table.at[src_v.at[j]],
                           rows_v.at[pl.ds(j * SUB, SUB)], sem)
          for j in range(NSUB)
      ]
      for d in descs:
        d.wait()
      for j in range(NSUB):
        pltpu.sync_copy(rows_v.at[pl.ds(j * SUB, SUB)], acc.at[dst_v.at[j]],
                        add=True)

    plsc.subcore_barrier()
    ro = pl.ds(s * ROWS_OUT, ROWS_OUT)
    pltpu.sync_copy(acc.at[ro], out_hbm.at[c, ro])

  return pl.kernel(
      body,
      out_type=jax.ShapeDtypeStruct((N_CORES, N_NODES, feats), jnp.float32),
      mesh=_MESH,
      scratch_types=[
          pltpu.VMEM((NSUB, SUB), jnp.int32),
          pltpu.VMEM((NSUB, SUB), jnp.int32),
          pltpu.VMEM((BLK, feats), jnp.float32),
          pltpu.VMEM_SHARED((N_ACC, feats), jnp.float32),
          pltpu.SemaphoreType.DMA,
      ],
  )


_conv_h = _make_conv(H_FEATS)
_conv_out = _make_conv(OUT_FEATS)


def _tc1_body(x_ref, wg_ref, we_ref, be_ref, h0_ref, h1_ref, g1_ref, dis_ref):
  xb = x_ref[...]
  logits = jnp.dot(xb, wg_ref[...], preferred_element_type=jnp.float32)
  m = jnp.max(logits, axis=1, keepdims=True)
  l0 = logits[:, 0:1]
  l1 = logits[:, 1:2]
  l2 = logits[:, 2:3]
  g0 = l0 >= m
  g1 = (l1 >= m) & ~g0
  g2 = (l2 >= m) & ~g0 & ~g1
  g3 = ~g0 & ~g1 & ~g2
  h = jnp.zeros((xb.shape[0], H_FEATS), jnp.float32)
  for k, gk in enumerate((g0, g1, g2, g3)):
    hk = jnp.dot(xb, we_ref[k], preferred_element_type=jnp.float32)
    hk = hk + be_ref[k:k + 1, :]
    h = h + gk.astype(jnp.float32) * hk
  deg = h0_ref[:, 0:1] + h1_ref[:, 0:1] + 1.0
  dis = lax.rsqrt(deg)
  g1_ref[...] = h * dis
  dis_ref[...] = dis


def _tc1(x, w_gate, W_experts, b_experts, h0, h1):
  grid = (N_NODES // BN,)
  return pl.pallas_call(
      _tc1_body,
      grid=grid,
      in_specs=[
          pl.BlockSpec((BN, IN_FEATS), lambda i: (i, 0)),
          pl.BlockSpec((IN_FEATS, 4), lambda i: (0, 0)),
          pl.BlockSpec((4, IN_FEATS, H_FEATS), lambda i: (0, 0, 0)),
          pl.BlockSpec((4, H_FEATS), lambda i: (0, 0)),
          pl.BlockSpec((BN, 16), lambda i: (i, 0)),
          pl.BlockSpec((BN, 16), lambda i: (i, 0)),
      ],
      out_specs=[
          pl.BlockSpec((BN, H_FEATS), lambda i: (i, 0)),
          pl.BlockSpec((BN, 1), lambda i: (i, 0)),
      ],
      out_shape=[
          jax.ShapeDtypeStruct((N_NODES, H_FEATS), jnp.float32),
          jax.ShapeDtypeStruct((N_NODES, 1), jnp.float32),
      ],
  )(x, w_gate, W_experts, b_experts, h0, h1)


def _tc2_body(p0_ref, p1_ref, dis_ref, b1_ref, w2_ref, g2_ref):
  dis = dis_ref[...]
  h = (p0_ref[...] + p1_ref[...]) * dis + b1_ref[...]
  h = jnp.maximum(h, 0.0)
  g2_ref[...] = jnp.dot(h, w2_ref[...], preferred_element_type=jnp.float32) * dis


def _tc2(p0, p1, dis, b1, w2):
  grid = (N_NODES // BN,)
  return pl.pallas_call(
      _tc2_body,
      grid=grid,
      in_specs=[
          pl.BlockSpec((BN, H_FEATS), lambda i: (i, 0)),
          pl.BlockSpec((BN, H_FEATS), lambda i: (i, 0)),
          pl.BlockSpec((BN, 1), lambda i: (i, 0)),
          pl.BlockSpec((1, H_FEATS), lambda i: (0, 0)),
          pl.BlockSpec((H_FEATS, OUT_FEATS), lambda i: (0, 0)),
      ],
      out_specs=pl.BlockSpec((BN, OUT_FEATS), lambda i: (i, 0)),
      out_shape=jax.ShapeDtypeStruct((N_NODES, OUT_FEATS), jnp.float32),
  )(p0, p1, dis, b1, w2)


def _tc3_body(q0_ref, q1_ref, dis_ref, b2_ref, out_ref):
  z = (q0_ref[...] + q1_ref[...]) * dis_ref[...] + b2_ref[...]
  m = jnp.max(z, axis=1, keepdims=True)
  zs = z - m
  out_ref[...] = zs - jnp.log(jnp.sum(jnp.exp(zs), axis=1, keepdims=True))


def _tc3(q0, q1, dis, b2):
  grid = (N_NODES // BN,)
  return pl.pallas_call(
      _tc3_body,
      grid=grid,
      in_specs=[
          pl.BlockSpec((BN, OUT_FEATS), lambda i: (i, 0)),
          pl.BlockSpec((BN, OUT_FEATS), lambda i: (i, 0)),
          pl.BlockSpec((BN, 1), lambda i: (i, 0)),
          pl.BlockSpec((1, OUT_FEATS), lambda i: (0, 0)),
      ],
      out_specs=pl.BlockSpec((BN, OUT_FEATS), lambda i: (i, 0)),
      out_shape=jax.ShapeDtypeStruct((N_NODES, OUT_FEATS), jnp.float32),
  )(q0, q1, dis, b2)


@jax.jit
def kernel(x, edge_index, w_gate, W_experts, b_experts, b1, W2, b2):
  pad = E_PAD - E_EDGES
  fill = jnp.full((pad,), N_NODES, jnp.int32)
  src2d = jnp.concatenate([edge_index[0], fill]).reshape(-1, SUB)
  dst2d = jnp.concatenate([edge_index[1], fill]).reshape(-1, SUB)

  hist = _degree(dst2d, jnp.ones((SUB, 16), jnp.float32),
                 jnp.zeros((N_ACC, 16), jnp.float32))
  g1, dis = _tc1(x, w_gate, W_experts, b_experts, hist[0], hist[1])

  g1p = jnp.concatenate(
      [g1, jnp.zeros((N_ACC - N_NODES, H_FEATS), jnp.float32)])
  p = _conv_h(src2d, dst2d, g1p, jnp.zeros((N_ACC, H_FEATS), jnp.float32))

  g2 = _tc2(p[0], p[1], dis, b1.reshape(1, -1), W2)
  g2p = jnp.concatenate(
      [g2, jnp.zeros((N_ACC - N_NODES, OUT_FEATS), jnp.float32)])
  q = _conv_out(src2d, dst2d, g2p, jnp.zeros((N_ACC, OUT_FEATS), jnp.float32))

  return _tc3(q[0], q[1], dis, b2.reshape(1, -1))


# speedup vs baseline: 13.3930x; 13.3930x over previous
import jax
import jax.numpy as jnp
from jax import lax
from jax.experimental import pallas as pl
from jax.experimental.pallas import tpu as pltpu
from jax.experimental.pallas import tpu_sc as plsc

N_NODES = 10000
E_EDGES = 320000
IN_FEATS = 128
H_FEATS = 128
OUT_FEATS = 64

N_CORES = 2
N_SUBCORES = 16
NW = N_CORES * N_SUBCORES
SUB = 128
NSUB = 8
BLK = SUB * NSUB
DEG_BLOCKS = 10
CONV_BLOCKS = 20
E_PAD = NW * DEG_BLOCKS * BLK
IDX_ROWS = E_PAD // SUB
N_ACC = 10112
ROWS_ACC = N_ACC // N_SUBCORES
BN = 1000

_MESH = plsc.VectorSubcoreMesh(core_axis_name="c", subcore_axis_name="s")


def _degree_body(dst2d, ones_hbm, zeros_hbm, out_hbm, idx_v, ones_v, acc):
  c = lax.axis_index("c")
  s = lax.axis_index("s")
  wid = c * N_SUBCORES + s
  ra = pl.ds(s * ROWS_ACC, ROWS_ACC)
  pltpu.sync_copy(zeros_hbm.at[ra], acc.at[ra])
  pltpu.sync_copy(ones_hbm, ones_v)
  plsc.subcore_barrier()
  base = wid * DEG_BLOCKS

  @pl.loop(0, DEG_BLOCKS)
  def _(b):
    row0 = (base + b) * NSUB
    pltpu.sync_copy(dst2d.at[pl.ds(row0, NSUB)], idx_v)
    for j in range(NSUB):
      pltpu.sync_copy(ones_v, acc.at[idx_v.at[j]], add=True)

  plsc.subcore_barrier()
  pltpu.sync_copy(acc.at[ra], out_hbm.at[c, ra])


_degree = pl.kernel(
    _degree_body,
    out_type=jax.ShapeDtypeStruct((N_CORES, N_ACC, 16), jnp.float32),
    mesh=_MESH,
    scratch_types=[
        pltpu.VMEM((NSUB, SUB), jnp.int32),
        pltpu.VMEM((SUB, 16), jnp.float32),
        pltpu.VMEM_SHARED((N_ACC, 16), jnp.float32),
    ],
    compiler_params=pltpu.CompilerParams(use_tc_tiling_on_sc=False),
)


def _make_conv(feats):
  fh = feats // 2

  def body(srcs, dst2d, table, out_hbm, src_v, dst_v, rows_v, acc, sem):
    c = lax.axis_index("c")
    s = lax.axis_index("s")
    ra = pl.ds(s * ROWS_ACC, ROWS_ACC)
    pltpu.sync_copy(table.at[pl.ds(c * N_ACC + s * ROWS_ACC, ROWS_ACC)],
                    acc.at[ra])
    plsc.subcore_barrier()
    src_base = c * IDX_ROWS + s * CONV_BLOCKS * NSUB
    dst_base = s * CONV_BLOCKS * NSUB

    @pl.loop(0, CONV_BLOCKS)
    def _(b):
      pltpu.sync_copy(srcs.at[pl.ds(src_base + b * NSUB, NSUB)], src_v)
      pltpu.sync_copy(dst2d.at[pl.ds(dst_base + b * NSUB, NSUB)], dst_v)
      descs = [
          pltpu.async_copy(table.at[src_v.at[j]],
                           rows_v.at[pl.ds(j * SUB, SUB)], sem)
          for j in range(NSUB)
      ]
      for d in descs:
        d.wait()
      for j in range(NSUB):
        pltpu.sync_copy(rows_v.at[pl.ds(j * SUB, SUB)],
                        acc.at[dst_v.at[j]], add=True)

    plsc.subcore_barrier()
    pltpu.sync_copy(acc.at[ra], out_hbm.at[c, ra])

  return pl.kernel(
      body,
      out_type=jax.ShapeDtypeStruct((N_CORES, N_ACC, fh), jnp.float32),
      mesh=_MESH,
      scratch_types=[
          pltpu.VMEM((NSUB, SUB), jnp.int32),
          pltpu.VMEM((NSUB, SUB), jnp.int32),
          pltpu.VMEM((BLK, fh), jnp.float32),
          pltpu.VMEM_SHARED((N_ACC, fh), jnp.float32),
          pltpu.SemaphoreType.DMA,
      ],
      compiler_params=pltpu.CompilerParams(use_tc_tiling_on_sc=False),
  )


_conv_h = _make_conv(H_FEATS)
_conv_out = _make_conv(OUT_FEATS)


def _tc1_body(x_ref, wg_ref, we_ref, be_ref, h0_ref, h1_ref, g1_ref, dis_ref):
  xb = x_ref[...]
  logits = jnp.dot(xb, wg_ref[...], preferred_element_type=jnp.float32)
  m = jnp.max(logits, axis=1, keepdims=True)
  l0 = logits[:, 0:1]
  l1 = logits[:, 1:2]
  l2 = logits[:, 2:3]
  g0 = l0 >= m
  g1 = (l1 >= m) & ~g0
  g2 = (l2 >= m) & ~g0 & ~g1
  g3 = ~g0 & ~g1 & ~g2
  h = jnp.zeros((xb.shape[0], H_FEATS), jnp.float32)
  for k, gk in enumerate((g0, g1, g2, g3)):
    hk = jnp.dot(xb, we_ref[k], preferred_element_type=jnp.float32)
    hk = hk + be_ref[k:k + 1, :]
    h = h + gk.astype(jnp.float32) * hk
  deg = h0_ref[:, 0:1] + h1_ref[:, 0:1] + 1.0
  dis = lax.rsqrt(deg)
  g1_ref[...] = h * dis
  dis_ref[...] = dis


def _tc1(x, w_gate, W_experts, b_experts, h0, h1):
  grid = (N_NODES // BN,)
  return pl.pallas_call(
      _tc1_body,
      grid=grid,
      in_specs=[
          pl.BlockSpec((BN, IN_FEATS), lambda i: (i, 0)),
          pl.BlockSpec((IN_FEATS, 4), lambda i: (0, 0)),
          pl.BlockSpec((4, IN_FEATS, H_FEATS), lambda i: (0, 0, 0)),
          pl.BlockSpec((4, H_FEATS), lambda i: (0, 0)),
          pl.BlockSpec((BN, 16), lambda i: (i, 0)),
          pl.BlockSpec((BN, 16), lambda i: (i, 0)),
      ],
      out_specs=[
          pl.BlockSpec((BN, H_FEATS), lambda i: (i, 0)),
          pl.BlockSpec((BN, 1), lambda i: (i, 0)),
      ],
      out_shape=[
          jax.ShapeDtypeStruct((N_NODES, H_FEATS), jnp.float32),
          jax.ShapeDtypeStruct((N_NODES, 1), jnp.float32),
      ],
  )(x, w_gate, W_experts, b_experts, h0, h1)


def _tc2_body(p0_ref, p1_ref, dis_ref, b1_ref, w2_ref, g2_ref):
  dis = dis_ref[...]
  h = jnp.concatenate([p0_ref[...], p1_ref[...]], axis=1) * dis + b1_ref[...]
  h = jnp.maximum(h, 0.0)
  g2_ref[...] = jnp.dot(h, w2_ref[...], preferred_element_type=jnp.float32) * dis


def _tc2(p0, p1, dis, b1, w2):
  grid = (N_NODES // BN,)
  return pl.pallas_call(
      _tc2_body,
      grid=grid,
      in_specs=[
          pl.BlockSpec((BN, H_FEATS // 2), lambda i: (i, 0)),
          pl.BlockSpec((BN, H_FEATS // 2), lambda i: (i, 0)),
          pl.BlockSpec((BN, 1), lambda i: (i, 0)),
          pl.BlockSpec((1, H_FEATS), lambda i: (0, 0)),
          pl.BlockSpec((H_FEATS, OUT_FEATS), lambda i: (0, 0)),
      ],
      out_specs=pl.BlockSpec((BN, OUT_FEATS), lambda i: (i, 0)),
      out_shape=jax.ShapeDtypeStruct((N_NODES, OUT_FEATS), jnp.float32),
  )(p0, p1, dis, b1, w2)


def _tc3_body(q0_ref, q1_ref, dis_ref, b2_ref, out_ref):
  z = jnp.concatenate([q0_ref[...], q1_ref[...]], axis=1)
  z = z * dis_ref[...] + b2_ref[...]
  m = jnp.max(z, axis=1, keepdims=True)
  zs = z - m
  out_ref[...] = zs - jnp.log(jnp.sum(jnp.exp(zs), axis=1, keepdims=True))


def _tc3(q0, q1, dis, b2):
  grid = (N_NODES // BN,)
  return pl.pallas_call(
      _tc3_body,
      grid=grid,
      in_specs=[
          pl.BlockSpec((BN, OUT_FEATS // 2), lambda i: (i, 0)),
          pl.BlockSpec((BN, OUT_FEATS // 2), lambda i: (i, 0)),
          pl.BlockSpec((BN, 1), lambda i: (i, 0)),
          pl.BlockSpec((1, OUT_FEATS), lambda i: (0, 0)),
      ],
      out_specs=pl.BlockSpec((BN, OUT_FEATS), lambda i: (i, 0)),
      out_shape=jax.ShapeDtypeStruct((N_NODES, OUT_FEATS), jnp.float32),
  )(q0, q1, dis, b2)


def _stack_halves(g, feats):
  fh = feats // 2
  t = jnp.zeros((2 * N_ACC, fh), jnp.float32)
  t = t.at[:N_NODES, :].set(g[:, :fh])
  t = t.at[N_ACC:N_ACC + N_NODES, :].set(g[:, fh:])
  return t


@jax.jit
def kernel(x, edge_index, w_gate, W_experts, b_experts, b1, W2, b2):
  pad = E_PAD - E_EDGES
  fill = jnp.full((pad,), N_NODES, jnp.int32)
  src2d = jnp.concatenate([edge_index[0], fill]).reshape(-1, SUB)
  dst2d = jnp.concatenate([edge_index[1], fill]).reshape(-1, SUB)
  srcs = jnp.concatenate([src2d, src2d + N_ACC])

  hist = _degree(dst2d, jnp.ones((SUB, 16), jnp.float32),
                 jnp.zeros((N_ACC, 16), jnp.float32))
  g1, dis = _tc1(x, w_gate, W_experts, b_experts,
                 hist[0, :N_NODES], hist[1, :N_NODES])

  p = _conv_h(srcs, dst2d, _stack_halves(g1, H_FEATS))
  g2 = _tc2(p[0, :N_NODES], p[1, :N_NODES], dis, b1.reshape(1, -1), W2)
  q = _conv_out(srcs, dst2d, _stack_halves(g2, OUT_FEATS))
  return _tc3(q[0, :N_NODES], q[1, :N_NODES], dis, b2.reshape(1, -1))


# speedup vs baseline: 13.6246x; 1.0173x over previous
import jax
import jax.numpy as jnp
from jax import lax
from jax.experimental import pallas as pl
from jax.experimental.pallas import tpu as pltpu
from jax.experimental.pallas import tpu_sc as plsc

N_NODES = 10000
E_EDGES = 320000
IN_FEATS = 128
H_FEATS = 128
OUT_FEATS = 64

N_CORES = 2
N_SUBCORES = 16
NW = N_CORES * N_SUBCORES
SUB = 128
NSUB = 8
BLK = SUB * NSUB
DEG_BLOCKS = 10
CONV_BLOCKS = 20
E_PAD = NW * DEG_BLOCKS * BLK
IDX_ROWS = E_PAD // SUB
N_ACC = 10112
ROWS_ACC = N_ACC // N_SUBCORES
BN = 1000

_MESH = plsc.VectorSubcoreMesh(core_axis_name="c", subcore_axis_name="s")


def _degree_body(dst2d, ones_hbm, zeros_hbm, out_hbm, idx_v, ones_v, acc):
  c = lax.axis_index("c")
  s = lax.axis_index("s")
  wid = c * N_SUBCORES + s
  ra = pl.ds(s * ROWS_ACC, ROWS_ACC)
  pltpu.sync_copy(zeros_hbm.at[ra], acc.at[ra])
  pltpu.sync_copy(ones_hbm, ones_v)
  plsc.subcore_barrier()
  base = wid * DEG_BLOCKS

  @pl.loop(0, DEG_BLOCKS)
  def _(b):
    row0 = (base + b) * NSUB
    pltpu.sync_copy(dst2d.at[pl.ds(row0, NSUB)], idx_v)
    for j in range(NSUB):
      pltpu.sync_copy(ones_v, acc.at[idx_v.at[j]], add=True)

  plsc.subcore_barrier()
  pltpu.sync_copy(acc.at[ra], out_hbm.at[c, ra])


_degree = pl.kernel(
    _degree_body,
    out_type=jax.ShapeDtypeStruct((N_CORES, N_ACC, 16), jnp.float32),
    mesh=_MESH,
    scratch_types=[
        pltpu.VMEM((NSUB, SUB), jnp.int32),
        pltpu.VMEM((SUB, 16), jnp.float32),
        pltpu.VMEM_SHARED((N_ACC, 16), jnp.float32),
    ],
    compiler_params=pltpu.CompilerParams(use_tc_tiling_on_sc=False),
)


def _make_conv(feats):
  fh = feats // 2

  def body(srcs, dsts, table, out_hbm, src_v, dst_v, rows_v, acc, sem):
    c = lax.axis_index("c")
    s = lax.axis_index("s")
    ra = pl.ds(s * ROWS_ACC, ROWS_ACC)
    pltpu.sync_copy(table.at[pl.ds(c * N_ACC + s * ROWS_ACC, ROWS_ACC)],
                    acc.at[ra])
    plsc.subcore_barrier()
    src_base = c * E_PAD + s * CONV_BLOCKS * BLK
    dst_base = s * CONV_BLOCKS * BLK

    @pl.loop(0, CONV_BLOCKS)
    def _(b):
      pltpu.sync_copy(srcs.at[pl.ds(src_base + b * BLK, BLK)], src_v)
      pltpu.sync_copy(dsts.at[pl.ds(dst_base + b * BLK, BLK)], dst_v)
      pltpu.async_copy(table.at[src_v], rows_v, sem).wait()
      pltpu.sync_copy(rows_v, acc.at[dst_v], add=True)

    plsc.subcore_barrier()
    pltpu.sync_copy(acc.at[ra], out_hbm.at[c, ra])

  return pl.kernel(
      body,
      out_type=jax.ShapeDtypeStruct((N_CORES, N_ACC, fh), jnp.float32),
      mesh=_MESH,
      scratch_types=[
          pltpu.VMEM((BLK,), jnp.int32),
          pltpu.VMEM((BLK,), jnp.int32),
          pltpu.VMEM((BLK, fh), jnp.float32),
          pltpu.VMEM_SHARED((N_ACC, fh), jnp.float32),
          pltpu.SemaphoreType.DMA,
      ],
      compiler_params=pltpu.CompilerParams(use_tc_tiling_on_sc=False),
  )


_conv_h = _make_conv(H_FEATS)
_conv_out = _make_conv(OUT_FEATS)


def _tc1_body(x_ref, wg_ref, we_ref, be_ref, h0_ref, h1_ref, g1_ref, dis_ref):
  xb = x_ref[...]
  logits = jnp.dot(xb, wg_ref[...], preferred_element_type=jnp.float32)
  m = jnp.max(logits, axis=1, keepdims=True)
  l0 = logits[:, 0:1]
  l1 = logits[:, 1:2]
  l2 = logits[:, 2:3]
  g0 = l0 >= m
  g1 = (l1 >= m) & ~g0
  g2 = (l2 >= m) & ~g0 & ~g1
  g3 = ~g0 & ~g1 & ~g2
  h = jnp.zeros((xb.shape[0], H_FEATS), jnp.float32)
  for k, gk in enumerate((g0, g1, g2, g3)):
    hk = jnp.dot(xb, we_ref[k], preferred_element_type=jnp.float32)
    hk = hk + be_ref[k:k + 1, :]
    h = h + gk.astype(jnp.float32) * hk
  deg = h0_ref[:, 0:1] + h1_ref[:, 0:1] + 1.0
  dis = lax.rsqrt(deg)
  g1_ref[...] = h * dis
  dis_ref[...] = dis


def _tc1(x, w_gate, W_experts, b_experts, h0, h1):
  grid = (N_NODES // BN,)
  return pl.pallas_call(
      _tc1_body,
      grid=grid,
      in_specs=[
          pl.BlockSpec((BN, IN_FEATS), lambda i: (i, 0)),
          pl.BlockSpec((IN_FEATS, 4), lambda i: (0, 0)),
          pl.BlockSpec((4, IN_FEATS, H_FEATS), lambda i: (0, 0, 0)),
          pl.BlockSpec((4, H_FEATS), lambda i: (0, 0)),
          pl.BlockSpec((BN, 16), lambda i: (i, 0)),
          pl.BlockSpec((BN, 16), lambda i: (i, 0)),
      ],
      out_specs=[
          pl.BlockSpec((BN, H_FEATS), lambda i: (i, 0)),
          pl.BlockSpec((BN, 1), lambda i: (i, 0)),
      ],
      out_shape=[
          jax.ShapeDtypeStruct((N_NODES, H_FEATS), jnp.float32),
          jax.ShapeDtypeStruct((N_NODES, 1), jnp.float32),
      ],
  )(x, w_gate, W_experts, b_experts, h0, h1)


def _tc2_body(p0_ref, p1_ref, dis_ref, b1_ref, w2_ref, g2_ref):
  dis = dis_ref[...]
  h = jnp.concatenate([p0_ref[...], p1_ref[...]], axis=1) * dis + b1_ref[...]
  h = jnp.maximum(h, 0.0)
  g2_ref[...] = jnp.dot(h, w2_ref[...], preferred_element_type=jnp.float32) * dis


def _tc2(p0, p1, dis, b1, w2):
  grid = (N_NODES // BN,)
  return pl.pallas_call(
      _tc2_body,
      grid=grid,
      in_specs=[
          pl.BlockSpec((BN, H_FEATS // 2), lambda i: (i, 0)),
          pl.BlockSpec((BN, H_FEATS // 2), lambda i: (i, 0)),
          pl.BlockSpec((BN, 1), lambda i: (i, 0)),
          pl.BlockSpec((1, H_FEATS), lambda i: (0, 0)),
          pl.BlockSpec((H_FEATS, OUT_FEATS), lambda i: (0, 0)),
      ],
      out_specs=pl.BlockSpec((BN, OUT_FEATS), lambda i: (i, 0)),
      out_shape=jax.ShapeDtypeStruct((N_NODES, OUT_FEATS), jnp.float32),
  )(p0, p1, dis, b1, w2)


def _tc3_body(q0_ref, q1_ref, dis_ref, b2_ref, out_ref):
  z = jnp.concatenate([q0_ref[...], q1_ref[...]], axis=1)
  z = z * dis_ref[...] + b2_ref[...]
  m = jnp.max(z, axis=1, keepdims=True)
  zs = z - m
  out_ref[...] = zs - jnp.log(jnp.sum(jnp.exp(zs), axis=1, keepdims=True))


def _tc3(q0, q1, dis, b2):
  grid = (N_NODES // BN,)
  return pl.pallas_call(
      _tc3_body,
      grid=grid,
      in_specs=[
          pl.BlockSpec((BN, OUT_FEATS // 2), lambda i: (i, 0)),
          pl.BlockSpec((BN, OUT_FEATS // 2), lambda i: (i, 0)),
          pl.BlockSpec((BN, 1), lambda i: (i, 0)),
          pl.BlockSpec((1, OUT_FEATS), lambda i: (0, 0)),
      ],
      out_specs=pl.BlockSpec((BN, OUT_FEATS), lambda i: (i, 0)),
      out_shape=jax.ShapeDtypeStruct((N_NODES, OUT_FEATS), jnp.float32),
  )(q0, q1, dis, b2)


def _stack_halves(g, feats):
  fh = feats // 2
  t = jnp.zeros((2 * N_ACC, fh), jnp.float32)
  t = t.at[:N_NODES, :].set(g[:, :fh])
  t = t.at[N_ACC:N_ACC + N_NODES, :].set(g[:, fh:])
  return t


@jax.jit
def kernel(x, edge_index, w_gate, W_experts, b_experts, b1, W2, b2):
  pad = E_PAD - E_EDGES
  fill = jnp.full((pad,), N_NODES, jnp.int32)
  src1d = jnp.concatenate([edge_index[0], fill])
  dst1d = jnp.concatenate([edge_index[1], fill])
  dst2d = dst1d.reshape(-1, SUB)
  srcs = jnp.concatenate([src1d, src1d + N_ACC])

  hist = _degree(dst2d, jnp.ones((SUB, 16), jnp.float32),
                 jnp.zeros((N_ACC, 16), jnp.float32))
  g1, dis = _tc1(x, w_gate, W_experts, b_experts,
                 hist[0, :N_NODES], hist[1, :N_NODES])

  p = _conv_h(srcs, dst1d, _stack_halves(g1, H_FEATS))
  g2 = _tc2(p[0, :N_NODES], p[1, :N_NODES], dis, b1.reshape(1, -1), W2)
  q = _conv_out(srcs, dst1d, _stack_halves(g2, OUT_FEATS))
  return _tc3(q[0, :N_NODES], q[1, :N_NODES], dis, b2.reshape(1, -1))


# speedup vs baseline: 15.1418x; 1.1114x over previous
import jax
import jax.numpy as jnp
from jax import lax
from jax.experimental import pallas as pl
from jax.experimental.pallas import tpu as pltpu
from jax.experimental.pallas import tpu_sc as plsc

N_NODES = 10000
E_EDGES = 320000
IN_FEATS = 128
H_FEATS = 128
OUT_FEATS = 64

N_CORES = 2
N_SUBCORES = 16
NW = N_CORES * N_SUBCORES
SUB = 128
NSUB = 8
BLK = SUB * NSUB
DEG_BLOCKS = 10
CONV_BLOCKS = 20
E_PAD = NW * DEG_BLOCKS * BLK
IDX_ROWS = E_PAD // SUB
N_ACC = 10112
ROWS_ACC = N_ACC // N_SUBCORES
BN = 1000

_MESH = plsc.VectorSubcoreMesh(core_axis_name="c", subcore_axis_name="s")


def _degree_body(dst2d, ones_hbm, zeros_hbm, out_hbm, idx_v, ones_v, acc):
  c = lax.axis_index("c")
  s = lax.axis_index("s")
  wid = c * N_SUBCORES + s
  ra = pl.ds(s * ROWS_ACC, ROWS_ACC)
  pltpu.sync_copy(zeros_hbm.at[ra], acc.at[ra])
  pltpu.sync_copy(ones_hbm, ones_v)
  plsc.subcore_barrier()
  base = wid * DEG_BLOCKS

  @pl.loop(0, DEG_BLOCKS)
  def _(b):
    row0 = (base + b) * NSUB
    pltpu.sync_copy(dst2d.at[pl.ds(row0, NSUB)], idx_v)
    for j in range(NSUB):
      pltpu.sync_copy(ones_v, acc.at[idx_v.at[j]], add=True)

  plsc.subcore_barrier()
  pltpu.sync_copy(acc.at[ra], out_hbm.at[c, ra])


_degree = pl.kernel(
    _degree_body,
    out_type=jax.ShapeDtypeStruct((N_CORES, N_ACC, 16), jnp.float32),
    mesh=_MESH,
    scratch_types=[
        pltpu.VMEM((NSUB, SUB), jnp.int32),
        pltpu.VMEM((SUB, 16), jnp.float32),
        pltpu.VMEM_SHARED((N_ACC, 16), jnp.float32),
    ],
    compiler_params=pltpu.CompilerParams(use_tc_tiling_on_sc=False),
)


def _make_conv(feats):
  fh = feats // 2
  blk = 512 if fh >= 64 else 1024
  n_blocks = E_PAD // (N_SUBCORES * blk)

  def body(srcs, dsts, table, out_hbm, src_v0, dst_v0, src_v1, dst_v1,
           rows0, rows1, acc, sg0, sg1):
    c = lax.axis_index("c")
    s = lax.axis_index("s")
    ra = pl.ds(s * ROWS_ACC, ROWS_ACC)
    pltpu.sync_copy(table.at[pl.ds(c * N_ACC + s * ROWS_ACC, ROWS_ACC)],
                    acc.at[ra])
    plsc.subcore_barrier()
    src_base = c * E_PAD + s * n_blocks * blk
    dst_base = s * n_blocks * blk

    def load_idx(b, sv, dv):
      pltpu.sync_copy(srcs.at[pl.ds(src_base + b * blk, blk)], sv)
      pltpu.sync_copy(dsts.at[pl.ds(dst_base + b * blk, blk)], dv)

    load_idx(0, src_v0, dst_v0)
    pltpu.async_copy(table.at[src_v0], rows0, sg0)
    nh = n_blocks // 2

    @pl.loop(0, nh)
    def _(g):
      load_idx(2 * g + 1, src_v1, dst_v1)
      pltpu.make_async_copy(table.at[src_v0], rows0, sg0).wait()
      pltpu.async_copy(table.at[src_v1], rows1, sg1)
      pltpu.sync_copy(rows0, acc.at[dst_v0], add=True)

      @pl.when(g < nh - 1)
      def _():
        load_idx(2 * g + 2, src_v0, dst_v0)
        pltpu.async_copy(table.at[src_v0], rows0, sg0)

      pltpu.make_async_copy(table.at[src_v1], rows1, sg1).wait()
      pltpu.sync_copy(rows1, acc.at[dst_v1], add=True)

    plsc.subcore_barrier()
    pltpu.sync_copy(acc.at[ra], out_hbm.at[c, ra])

  return pl.kernel(
      body,
      out_type=jax.ShapeDtypeStruct((N_CORES, N_ACC, fh), jnp.float32),
      mesh=_MESH,
      scratch_types=[
          pltpu.VMEM((blk,), jnp.int32),
          pltpu.VMEM((blk,), jnp.int32),
          pltpu.VMEM((blk,), jnp.int32),
          pltpu.VMEM((blk,), jnp.int32),
          pltpu.VMEM((blk, fh), jnp.float32),
          pltpu.VMEM((blk, fh), jnp.float32),
          pltpu.VMEM_SHARED((N_ACC, fh), jnp.float32),
          pltpu.SemaphoreType.DMA,
          pltpu.SemaphoreType.DMA,
      ],
      compiler_params=pltpu.CompilerParams(use_tc_tiling_on_sc=False),
  )


_conv_h = _make_conv(H_FEATS)
_conv_out = _make_conv(OUT_FEATS)


def _tc1_body(x_ref, wg_ref, we_ref, be_ref, h0_ref, h1_ref, g1_ref, dis_ref):
  xb = x_ref[...]
  logits = jnp.dot(xb, wg_ref[...], preferred_element_type=jnp.float32)
  m = jnp.max(logits, axis=1, keepdims=True)
  l0 = logits[:, 0:1]
  l1 = logits[:, 1:2]
  l2 = logits[:, 2:3]
  g0 = l0 >= m
  g1 = (l1 >= m) & ~g0
  g2 = (l2 >= m) & ~g0 & ~g1
  g3 = ~g0 & ~g1 & ~g2
  h = jnp.zeros((xb.shape[0], H_FEATS), jnp.float32)
  for k, gk in enumerate((g0, g1, g2, g3)):
    hk = jnp.dot(xb, we_ref[k], preferred_element_type=jnp.float32)
    hk = hk + be_ref[k:k + 1, :]
    h = h + gk.astype(jnp.float32) * hk
  deg = h0_ref[:, 0:1] + h1_ref[:, 0:1] + 1.0
  dis = lax.rsqrt(deg)
  g1_ref[...] = h * dis
  dis_ref[...] = dis


def _tc1(x, w_gate, W_experts, b_experts, h0, h1):
  grid = (N_NODES // BN,)
  return pl.pallas_call(
      _tc1_body,
      grid=grid,
      in_specs=[
          pl.BlockSpec((BN, IN_FEATS), lambda i: (i, 0)),
          pl.BlockSpec((IN_FEATS, 4), lambda i: (0, 0)),
          pl.BlockSpec((4, IN_FEATS, H_FEATS), lambda i: (0, 0, 0)),
          pl.BlockSpec((4, H_FEATS), lambda i: (0, 0)),
          pl.BlockSpec((BN, 16), lambda i: (i, 0)),
          pl.BlockSpec((BN, 16), lambda i: (i, 0)),
      ],
      out_specs=[
          pl.BlockSpec((BN, H_FEATS), lambda i: (i, 0)),
          pl.BlockSpec((BN, 1), lambda i: (i, 0)),
      ],
      out_shape=[
          jax.ShapeDtypeStruct((N_NODES, H_FEATS), jnp.float32),
          jax.ShapeDtypeStruct((N_NODES, 1), jnp.float32),
      ],
  )(x, w_gate, W_experts, b_experts, h0, h1)


def _tc2_body(p0_ref, p1_ref, dis_ref, b1_ref, w2_ref, g2_ref):
  dis = dis_ref[...]
  h = jnp.concatenate([p0_ref[...], p1_ref[...]], axis=1) * dis + b1_ref[...]
  h = jnp.maximum(h, 0.0)
  g2_ref[...] = jnp.dot(h, w2_ref[...], preferred_element_type=jnp.float32) * dis


def _tc2(p0, p1, dis, b1, w2):
  grid = (N_NODES // BN,)
  return pl.pallas_call(
      _tc2_body,
      grid=grid,
      in_specs=[
          pl.BlockSpec((BN, H_FEATS // 2), lambda i: (i, 0)),
          pl.BlockSpec((BN, H_FEATS // 2), lambda i: (i, 0)),
          pl.BlockSpec((BN, 1), lambda i: (i, 0)),
          pl.BlockSpec((1, H_FEATS), lambda i: (0, 0)),
          pl.BlockSpec((H_FEATS, OUT_FEATS), lambda i: (0, 0)),
      ],
      out_specs=pl.BlockSpec((BN, OUT_FEATS), lambda i: (i, 0)),
      out_shape=jax.ShapeDtypeStruct((N_NODES, OUT_FEATS), jnp.float32),
  )(p0, p1, dis, b1, w2)


def _tc3_body(q0_ref, q1_ref, dis_ref, b2_ref, out_ref):
  z = jnp.concatenate([q0_ref[...], q1_ref[...]], axis=1)
  z = z * dis_ref[...] + b2_ref[...]
  m = jnp.max(z, axis=1, keepdims=True)
  zs = z - m
  out_ref[...] = zs - jnp.log(jnp.sum(jnp.exp(zs), axis=1, keepdims=True))


def _tc3(q0, q1, dis, b2):
  grid = (N_NODES // BN,)
  return pl.pallas_call(
      _tc3_body,
      grid=grid,
      in_specs=[
          pl.BlockSpec((BN, OUT_FEATS // 2), lambda i: (i, 0)),
          pl.BlockSpec((BN, OUT_FEATS // 2), lambda i: (i, 0)),
          pl.BlockSpec((BN, 1), lambda i: (i, 0)),
          pl.BlockSpec((1, OUT_FEATS), lambda i: (0, 0)),
      ],
      out_specs=pl.BlockSpec((BN, OUT_FEATS), lambda i: (i, 0)),
      out_shape=jax.ShapeDtypeStruct((N_NODES, OUT_FEATS), jnp.float32),
  )(q0, q1, dis, b2)


def _stack_halves(g, feats):
  fh = feats // 2
  t = jnp.zeros((2 * N_ACC, fh), jnp.float32)
  t = t.at[:N_NODES, :].set(g[:, :fh])
  t = t.at[N_ACC:N_ACC + N_NODES, :].set(g[:, fh:])
  return t


@jax.jit
def kernel(x, edge_index, w_gate, W_experts, b_experts, b1, W2, b2):
  pad = E_PAD - E_EDGES
  fill = jnp.full((pad,), N_NODES, jnp.int32)
  src1d = jnp.concatenate([edge_index[0], fill])
  dst1d = jnp.concatenate([edge_index[1], fill])
  dst2d = dst1d.reshape(-1, SUB)
  srcs = jnp.concatenate([src1d, src1d + N_ACC])

  hist = _degree(dst2d, jnp.ones((SUB, 16), jnp.float32),
                 jnp.zeros((N_ACC, 16), jnp.float32))
  g1, dis = _tc1(x, w_gate, W_experts, b_experts,
                 hist[0, :N_NODES], hist[1, :N_NODES])

  p = _conv_h(srcs, dst1d, _stack_halves(g1, H_FEATS))
  g2 = _tc2(p[0, :N_NODES], p[1, :N_NODES], dis, b1.reshape(1, -1), W2)
  q = _conv_out(srcs, dst1d, _stack_halves(g2, OUT_FEATS))
  return _tc3(q[0, :N_NODES], q[1, :N_NODES], dis, b2.reshape(1, -1))


# speedup vs baseline: 20.4638x; 1.3515x over previous
import jax
import jax.numpy as jnp
from jax import lax
from jax.experimental import pallas as pl
from jax.experimental.pallas import tpu as pltpu
from jax.experimental.pallas import tpu_sc as plsc

N_NODES = 10000
E_EDGES = 320000
IN_FEATS = 128
H_FEATS = 128
OUT_FEATS = 64

N_CORES = 2
N_SUBCORES = 16
NW = N_CORES * N_SUBCORES
SUB = 128
NSUB = 8
BLK = SUB * NSUB
DEG_BLOCKS = 10
CONV_BLOCKS = 20
E_PAD = NW * DEG_BLOCKS * BLK
IDX_ROWS = E_PAD // SUB
N_ACC = 10112
ROWS_ACC = N_ACC // N_SUBCORES
BN = 1000

_MESH = plsc.VectorSubcoreMesh(core_axis_name="c", subcore_axis_name="s")


def _degree_body(dst2d, ones_hbm, zeros_hbm, out_hbm, idx_v, ones_v, acc):
  c = lax.axis_index("c")
  s = lax.axis_index("s")
  wid = c * N_SUBCORES + s
  ra = pl.ds(s * ROWS_ACC, ROWS_ACC)
  pltpu.sync_copy(zeros_hbm.at[ra], acc.at[ra])
  pltpu.sync_copy(ones_hbm, ones_v)
  plsc.subcore_barrier()
  base = wid * DEG_BLOCKS

  @pl.loop(0, DEG_BLOCKS)
  def _(b):
    row0 = (base + b) * NSUB
    pltpu.sync_copy(dst2d.at[pl.ds(row0, NSUB)], idx_v)
    for j in range(NSUB):
      pltpu.sync_copy(ones_v, acc.at[idx_v.at[j]], add=True)

  plsc.subcore_barrier()
  pltpu.sync_copy(acc.at[ra], out_hbm.at[c, ra])


_degree = pl.kernel(
    _degree_body,
    out_type=jax.ShapeDtypeStruct((N_CORES, N_ACC, 16), jnp.float32),
    mesh=_MESH,
    scratch_types=[
        pltpu.VMEM((NSUB, SUB), jnp.int32),
        pltpu.VMEM((SUB, 16), jnp.float32),
        pltpu.VMEM_SHARED((N_ACC, 16), jnp.float32),
    ],
    compiler_params=pltpu.CompilerParams(use_tc_tiling_on_sc=False),
)


def _make_conv(feats):
  fh = feats // 2
  blk = 256 if fh >= 64 else 1024
  n_blocks = E_PAD // (N_SUBCORES * blk)

  def body(srcs, dsts, table, out_hbm, src_v0, dst_v0, src_v1, dst_v1,
           rows0, rows1, tab_sp, acc, sg0, sg1):
    c = lax.axis_index("c")
    s = lax.axis_index("s")
    ra = pl.ds(s * ROWS_ACC, ROWS_ACC)
    th = pl.ds(c * N_ACC + s * ROWS_ACC, ROWS_ACC)
    pltpu.sync_copy(table.at[th], tab_sp.at[ra])
    pltpu.sync_copy(table.at[th], acc.at[ra])
    plsc.subcore_barrier()
    src_base = s * n_blocks * blk
    dst_base = s * n_blocks * blk

    def load_idx(b, sv, dv):
      pltpu.sync_copy(srcs.at[pl.ds(src_base + b * blk, blk)], sv)
      pltpu.sync_copy(dsts.at[pl.ds(dst_base + b * blk, blk)], dv)

    load_idx(0, src_v0, dst_v0)
    pltpu.async_copy(tab_sp.at[src_v0], rows0, sg0)
    nh = n_blocks // 2

    @pl.loop(0, nh)
    def _(g):
      load_idx(2 * g + 1, src_v1, dst_v1)
      pltpu.make_async_copy(tab_sp.at[src_v0], rows0, sg0).wait()
      pltpu.async_copy(tab_sp.at[src_v1], rows1, sg1)
      pltpu.sync_copy(rows0, acc.at[dst_v0], add=True)

      @pl.when(g < nh - 1)
      def _():
        load_idx(2 * g + 2, src_v0, dst_v0)
        pltpu.async_copy(tab_sp.at[src_v0], rows0, sg0)

      pltpu.make_async_copy(tab_sp.at[src_v1], rows1, sg1).wait()
      pltpu.sync_copy(rows1, acc.at[dst_v1], add=True)

    plsc.subcore_barrier()
    pltpu.sync_copy(acc.at[ra], out_hbm.at[c, ra])

  return pl.kernel(
      body,
      out_type=jax.ShapeDtypeStruct((N_CORES, N_ACC, fh), jnp.float32),
      mesh=_MESH,
      scratch_types=[
          pltpu.VMEM((blk,), jnp.int32),
          pltpu.VMEM((blk,), jnp.int32),
          pltpu.VMEM((blk,), jnp.int32),
          pltpu.VMEM((blk,), jnp.int32),
          pltpu.VMEM((blk, fh), jnp.float32),
          pltpu.VMEM((blk, fh), jnp.float32),
          pltpu.VMEM_SHARED((N_ACC, fh), jnp.float32),
          pltpu.VMEM_SHARED((N_ACC, fh), jnp.float32),
          pltpu.SemaphoreType.DMA,
          pltpu.SemaphoreType.DMA,
      ],
      compiler_params=pltpu.CompilerParams(use_tc_tiling_on_sc=False),
  )


_conv_h = _make_conv(H_FEATS)
_conv_out = _make_conv(OUT_FEATS)


def _tc1_body(x_ref, wg_ref, we_ref, be_ref, h0_ref, h1_ref, g1_ref, dis_ref):
  xb = x_ref[...]
  logits = jnp.dot(xb, wg_ref[...], preferred_element_type=jnp.float32)
  m = jnp.max(logits, axis=1, keepdims=True)
  l0 = logits[:, 0:1]
  l1 = logits[:, 1:2]
  l2 = logits[:, 2:3]
  g0 = l0 >= m
  g1 = (l1 >= m) & ~g0
  g2 = (l2 >= m) & ~g0 & ~g1
  g3 = ~g0 & ~g1 & ~g2
  h = jnp.zeros((xb.shape[0], H_FEATS), jnp.float32)
  for k, gk in enumerate((g0, g1, g2, g3)):
    hk = jnp.dot(xb, we_ref[k], preferred_element_type=jnp.float32)
    hk = hk + be_ref[k:k + 1, :]
    h = h + gk.astype(jnp.float32) * hk
  deg = h0_ref[:, 0:1] + h1_ref[:, 0:1] + 1.0
  dis = lax.rsqrt(deg)
  g1_ref[...] = h * dis
  dis_ref[...] = dis


def _tc1(x, w_gate, W_experts, b_experts, h0, h1):
  grid = (N_NODES // BN,)
  return pl.pallas_call(
      _tc1_body,
      grid=grid,
      in_specs=[
          pl.BlockSpec((BN, IN_FEATS), lambda i: (i, 0)),
          pl.BlockSpec((IN_FEATS, 4), lambda i: (0, 0)),
          pl.BlockSpec((4, IN_FEATS, H_FEATS), lambda i: (0, 0, 0)),
          pl.BlockSpec((4, H_FEATS), lambda i: (0, 0)),
          pl.BlockSpec((BN, 16), lambda i: (i, 0)),
          pl.BlockSpec((BN, 16), lambda i: (i, 0)),
      ],
      out_specs=[
          pl.BlockSpec((BN, H_FEATS), lambda i: (i, 0)),
          pl.BlockSpec((BN, 1), lambda i: (i, 0)),
      ],
      out_shape=[
          jax.ShapeDtypeStruct((N_NODES, H_FEATS), jnp.float32),
          jax.ShapeDtypeStruct((N_NODES, 1), jnp.float32),
      ],
  )(x, w_gate, W_experts, b_experts, h0, h1)


def _tc2_body(p0_ref, p1_ref, dis_ref, b1_ref, w2_ref, g2_ref):
  dis = dis_ref[...]
  h = jnp.concatenate([p0_ref[...], p1_ref[...]], axis=1) * dis + b1_ref[...]
  h = jnp.maximum(h, 0.0)
  g2_ref[...] = jnp.dot(h, w2_ref[...], preferred_element_type=jnp.float32) * dis


def _tc2(p0, p1, dis, b1, w2):
  grid = (N_NODES // BN,)
  return pl.pallas_call(
      _tc2_body,
      grid=grid,
      in_specs=[
          pl.BlockSpec((BN, H_FEATS // 2), lambda i: (i, 0)),
          pl.BlockSpec((BN, H_FEATS // 2), lambda i: (i, 0)),
          pl.BlockSpec((BN, 1), lambda i: (i, 0)),
          pl.BlockSpec((1, H_FEATS), lambda i: (0, 0)),
          pl.BlockSpec((H_FEATS, OUT_FEATS), lambda i: (0, 0)),
      ],
      out_specs=pl.BlockSpec((BN, OUT_FEATS), lambda i: (i, 0)),
      out_shape=jax.ShapeDtypeStruct((N_NODES, OUT_FEATS), jnp.float32),
  )(p0, p1, dis, b1, w2)


def _tc3_body(q0_ref, q1_ref, dis_ref, b2_ref, out_ref):
  z = jnp.concatenate([q0_ref[...], q1_ref[...]], axis=1)
  z = z * dis_ref[...] + b2_ref[...]
  m = jnp.max(z, axis=1, keepdims=True)
  zs = z - m
  out_ref[...] = zs - jnp.log(jnp.sum(jnp.exp(zs), axis=1, keepdims=True))


def _tc3(q0, q1, dis, b2):
  grid = (N_NODES // BN,)
  return pl.pallas_call(
      _tc3_body,
      grid=grid,
      in_specs=[
          pl.BlockSpec((BN, OUT_FEATS // 2), lambda i: (i, 0)),
          pl.BlockSpec((BN, OUT_FEATS // 2), lambda i: (i, 0)),
          pl.BlockSpec((BN, 1), lambda i: (i, 0)),
          pl.BlockSpec((1, OUT_FEATS), lambda i: (0, 0)),
      ],
      out_specs=pl.BlockSpec((BN, OUT_FEATS), lambda i: (i, 0)),
      out_shape=jax.ShapeDtypeStruct((N_NODES, OUT_FEATS), jnp.float32),
  )(q0, q1, dis, b2)


def _stack_halves(g, feats):
  fh = feats // 2
  t = jnp.zeros((2 * N_ACC, fh), jnp.float32)
  t = t.at[:N_NODES, :].set(g[:, :fh])
  t = t.at[N_ACC:N_ACC + N_NODES, :].set(g[:, fh:])
  return t


@jax.jit
def kernel(x, edge_index, w_gate, W_experts, b_experts, b1, W2, b2):
  pad = E_PAD - E_EDGES
  fill = jnp.full((pad,), N_NODES, jnp.int32)
  src1d = jnp.concatenate([edge_index[0], fill])
  dst1d = jnp.concatenate([edge_index[1], fill])
  dst2d = dst1d.reshape(-1, SUB)

  hist = _degree(dst2d, jnp.ones((SUB, 16), jnp.float32),
                 jnp.zeros((N_ACC, 16), jnp.float32))
  g1, dis = _tc1(x, w_gate, W_experts, b_experts,
                 hist[0, :N_NODES], hist[1, :N_NODES])

  p = _conv_h(src1d, dst1d, _stack_halves(g1, H_FEATS))
  g2 = _tc2(p[0, :N_NODES], p[1, :N_NODES], dis, b1.reshape(1, -1), W2)
  q = _conv_out(src1d, dst1d, _stack_halves(g2, OUT_FEATS))
  return _tc3(q[0, :N_NODES], q[1, :N_NODES], dis, b2.reshape(1, -1))


# speedup vs baseline: 22.4730x; 1.0982x over previous
import jax
import jax.numpy as jnp
from jax import lax
from jax.experimental import pallas as pl
from jax.experimental.pallas import tpu as pltpu
from jax.experimental.pallas import tpu_sc as plsc

N_NODES = 10000
E_EDGES = 320000
IN_FEATS = 128
H_FEATS = 128
OUT_FEATS = 64

N_CORES = 2
N_SUBCORES = 16
N_ACC = 10112
ROWS_ACC = N_ACC // N_SUBCORES
BN = 1000

DEG_BLK = 1000
DEG_BLOCKS = E_EDGES // (N_CORES * N_SUBCORES * DEG_BLK)

_MESH = plsc.VectorSubcoreMesh(core_axis_name="c", subcore_axis_name="s")
_SC_PARAMS = pltpu.CompilerParams(use_tc_tiling_on_sc=False)


def _degree_body(edges, ones_hbm, zeros_hbm, out_hbm, idx_v, ones_v, acc):
  c = lax.axis_index("c")
  s = lax.axis_index("s")
  wid = c * N_SUBCORES + s
  ra = pl.ds(s * ROWS_ACC, ROWS_ACC)
  pltpu.sync_copy(zeros_hbm.at[ra], acc.at[ra])
  pltpu.sync_copy(ones_hbm, ones_v)
  plsc.subcore_barrier()
  base = wid * DEG_BLOCKS * DEG_BLK

  @pl.loop(0, DEG_BLOCKS)
  def _(b):
    pltpu.sync_copy(edges.at[1, pl.ds(base + b * DEG_BLK, DEG_BLK)], idx_v)
    pltpu.sync_copy(ones_v, acc.at[idx_v], add=True)

  plsc.subcore_barrier()
  pltpu.sync_copy(acc.at[ra], out_hbm.at[c, ra])


_degree = pl.kernel(
    _degree_body,
    out_type=jax.ShapeDtypeStruct((N_CORES, N_ACC, 16), jnp.float32),
    mesh=_MESH,
    scratch_types=[
        pltpu.VMEM((DEG_BLK,), jnp.int32),
        pltpu.VMEM((DEG_BLK, 16), jnp.float32),
        pltpu.VMEM_SHARED((N_ACC, 16), jnp.float32),
    ],
    compiler_params=_SC_PARAMS,
)


def _make_conv(feats, blk):
  fh = feats // 2
  n_blocks = E_EDGES // (N_SUBCORES * blk)

  def body(edges, ta, tb, out_hbm, src_v0, dst_v0, src_v1, dst_v1,
           rows0, rows1, tab_sp, acc, sg0, sg1):
    c = lax.axis_index("c")
    s = lax.axis_index("s")
    ra = pl.ds(s * ROWS_ACC, ROWS_ACC)

    @pl.when(c == 0)
    def _():
      pltpu.sync_copy(ta.at[ra], tab_sp.at[ra])
      pltpu.sync_copy(ta.at[ra], acc.at[ra])

    @pl.when(c != 0)
    def _():
      pltpu.sync_copy(tb.at[ra], tab_sp.at[ra])
      pltpu.sync_copy(tb.at[ra], acc.at[ra])

    plsc.subcore_barrier()
    base = s * n_blocks * blk

    def load_idx(b, sv, dv):
      pltpu.sync_copy(edges.at[0, pl.ds(base + b * blk, blk)], sv)
      pltpu.sync_copy(edges.at[1, pl.ds(base + b * blk, blk)], dv)

    load_idx(0, src_v0, dst_v0)
    pltpu.async_copy(tab_sp.at[src_v0], rows0, sg0)
    nh = n_blocks // 2

    @pl.loop(0, nh)
    def _(g):
      load_idx(2 * g + 1, src_v1, dst_v1)
      pltpu.make_async_copy(tab_sp.at[src_v0], rows0, sg0).wait()
      pltpu.async_copy(tab_sp.at[src_v1], rows1, sg1)
      pltpu.sync_copy(rows0, acc.at[dst_v0], add=True)

      @pl.when(g < nh - 1)
      def _():
        load_idx(2 * g + 2, src_v0, dst_v0)
        pltpu.async_copy(tab_sp.at[src_v0], rows0, sg0)

      pltpu.make_async_copy(tab_sp.at[src_v1], rows1, sg1).wait()
      pltpu.sync_copy(rows1, acc.at[dst_v1], add=True)

    plsc.subcore_barrier()
    pltpu.sync_copy(acc.at[ra], out_hbm.at[c, ra])

  return pl.kernel(
      body,
      out_type=jax.ShapeDtypeStruct((N_CORES, N_ACC, fh), jnp.float32),
      mesh=_MESH,
      scratch_types=[
          pltpu.VMEM((blk,), jnp.int32),
          pltpu.VMEM((blk,), jnp.int32),
          pltpu.VMEM((blk,), jnp.int32),
          pltpu.VMEM((blk,), jnp.int32),
          pltpu.VMEM((blk, fh), jnp.float32),
          pltpu.VMEM((blk, fh), jnp.float32),
          pltpu.VMEM_SHARED((N_ACC, fh), jnp.float32),
          pltpu.VMEM_SHARED((N_ACC, fh), jnp.float32),
          pltpu.SemaphoreType.DMA,
          pltpu.SemaphoreType.DMA,
      ],
      compiler_params=_SC_PARAMS,
  )


_conv_h = _make_conv(H_FEATS, 200)
_conv_out = _make_conv(OUT_FEATS, 1000)


def _tc1_body(x_ref, wg_ref, we_ref, be_ref, h0_ref, h1_ref,
              ta_ref, tb_ref, dis_ref):
  xb = x_ref[...]
  logits = jnp.dot(xb, wg_ref[...], preferred_element_type=jnp.float32)
  m = jnp.max(logits, axis=1, keepdims=True)
  l0 = logits[:, 0:1]
  l1 = logits[:, 1:2]
  l2 = logits[:, 2:3]
  g0 = l0 >= m
  g1 = (l1 >= m) & ~g0
  g2 = (l2 >= m) & ~g0 & ~g1
  g3 = ~g0 & ~g1 & ~g2
  h = jnp.zeros((xb.shape[0], H_FEATS), jnp.float32)
  for k, gk in enumerate((g0, g1, g2, g3)):
    hk = jnp.dot(xb, we_ref[k], preferred_element_type=jnp.float32)
    hk = hk + be_ref[k:k + 1, :]
    h = h + gk.astype(jnp.float32) * hk
  deg = h0_ref[0, :, 0:1] + h1_ref[0, :, 0:1] + 1.0
  dis = lax.rsqrt(deg)
  g1s = h * dis
  ta_ref[...] = g1s[:, :H_FEATS // 2]
  tb_ref[...] = g1s[:, H_FEATS // 2:]
  dis_ref[...] = dis


def _tc1(x, w_gate, W_experts, b_experts, hist):
  grid = (N_NODES // BN,)
  return pl.pallas_call(
      _tc1_body,
      grid=grid,
      in_specs=[
          pl.BlockSpec((BN, IN_FEATS), lambda i: (i, 0)),
          pl.BlockSpec((IN_FEATS, 4), lambda i: (0, 0)),
          pl.BlockSpec((4, IN_FEATS, H_FEATS), lambda i: (0, 0, 0)),
          pl.BlockSpec((4, H_FEATS), lambda i: (0, 0)),
          pl.BlockSpec((1, BN, 16), lambda i: (0, i, 0)),
          pl.BlockSpec((1, BN, 16), lambda i: (1, i, 0)),
      ],
      out_specs=[
          pl.BlockSpec((BN, H_FEATS // 2), lambda i: (i, 0)),
          pl.BlockSpec((BN, H_FEATS // 2), lambda i: (i, 0)),
          pl.BlockSpec((BN, 1), lambda i: (i, 0)),
      ],
      out_shape=[
          jax.ShapeDtypeStruct((N_ACC, H_FEATS // 2), jnp.float32),
          jax.ShapeDtypeStruct((N_ACC, H_FEATS // 2), jnp.float32),
          jax.ShapeDtypeStruct((N_NODES, 1), jnp.float32),
      ],
  )(x, w_gate, W_experts, b_experts, hist, hist)


def _tc2_body(p0_ref, p1_ref, dis_ref, b1_ref, w2_ref, ta_ref, tb_ref):
  dis = dis_ref[...]
  h = jnp.concatenate([p0_ref[0], p1_ref[0]], axis=1) * dis + b1_ref[...]
  h = jnp.maximum(h, 0.0)
  g2 = jnp.dot(h, w2_ref[...], preferred_element_type=jnp.float32) * dis
  ta_ref[...] = g2[:, :OUT_FEATS // 2]
  tb_ref[...] = g2[:, OUT_FEATS // 2:]


def _tc2(p, dis, b1, w2):
  grid = (N_NODES // BN,)
  return pl.pallas_call(
      _tc2_body,
      grid=grid,
      in_specs=[
          pl.BlockSpec((1, BN, H_FEATS // 2), lambda i: (0, i, 0)),
          pl.BlockSpec((1, BN, H_FEATS // 2), lambda i: (1, i, 0)),
          pl.BlockSpec((BN, 1), lambda i: (i, 0)),
          pl.BlockSpec((1, H_FEATS), lambda i: (0, 0)),
          pl.BlockSpec((H_FEATS, OUT_FEATS), lambda i: (0, 0)),
      ],
      out_specs=[
          pl.BlockSpec((BN, OUT_FEATS // 2), lambda i: (i, 0)),
          pl.BlockSpec((BN, OUT_FEATS // 2), lambda i: (i, 0)),
      ],
      out_shape=[
          jax.ShapeDtypeStruct((N_ACC, OUT_FEATS // 2), jnp.float32),
          jax.ShapeDtypeStruct((N_ACC, OUT_FEATS // 2), jnp.float32),
      ],
  )(p, p, dis, b1, w2)


def _tc3_body(q0_ref, q1_ref, dis_ref, b2_ref, out_ref):
  z = jnp.concatenate([q0_ref[0], q1_ref[0]], axis=1)
  z = z * dis_ref[...] + b2_ref[...]
  m = jnp.max(z, axis=1, keepdims=True)
  zs = z - m
  out_ref[...] = zs - jnp.log(jnp.sum(jnp.exp(zs), axis=1, keepdims=True))


def _tc3(q, dis, b2):
  grid = (N_NODES // BN,)
  return pl.pallas_call(
      _tc3_body,
      grid=grid,
      in_specs=[
          pl.BlockSpec((1, BN, OUT_FEATS // 2), lambda i: (0, i, 0)),
          pl.BlockSpec((1, BN, OUT_FEATS // 2), lambda i: (1, i, 0)),
          pl.BlockSpec((BN, 1), lambda i: (i, 0)),
          pl.BlockSpec((1, OUT_FEATS), lambda i: (0, 0)),
      ],
      out_specs=pl.BlockSpec((BN, OUT_FEATS), lambda i: (i, 0)),
      out_shape=jax.ShapeDtypeStruct((N_NODES, OUT_FEATS), jnp.float32),
  )(q, q, dis, b2)


@jax.jit
def kernel(x, edge_index, w_gate, W_experts, b_experts, b1, W2, b2):
  hist = _degree(edge_index, jnp.ones((DEG_BLK, 16), jnp.float32),
                 jnp.zeros((N_ACC, 16), jnp.float32))
  ta, tb, dis = _tc1(x, w_gate, W_experts, b_experts, hist)
  p = _conv_h(edge_index, ta, tb)
  t2a, t2b = _tc2(p, dis, b1.reshape(1, -1), W2)
  q = _conv_out(edge_index, t2a, t2b)
  return _tc3(q, dis, b2.reshape(1, -1))


# speedup vs baseline: 23.8665x; 1.0620x over previous
import jax
import jax.numpy as jnp
from jax import lax
from jax.experimental import pallas as pl
from jax.experimental.pallas import tpu as pltpu
from jax.experimental.pallas import tpu_sc as plsc

N_NODES = 10000
E_EDGES = 320000
IN_FEATS = 128
H_FEATS = 128
OUT_FEATS = 64

N_CORES = 2
N_SUBCORES = 16
N_ACC = 10112
ROWS_ACC = N_ACC // N_SUBCORES
BN = 1000

DEG_BLK = 1000
DEG_BLOCKS = E_EDGES // (N_CORES * N_SUBCORES * DEG_BLK)

_MESH = plsc.VectorSubcoreMesh(core_axis_name="c", subcore_axis_name="s")
_SC_PARAMS = pltpu.CompilerParams(use_tc_tiling_on_sc=False)


def _degree_body(edges, ones_hbm, zeros_hbm, out_hbm, idx_v, ones_v, acc):
  c = lax.axis_index("c")
  s = lax.axis_index("s")
  wid = c * N_SUBCORES + s
  ra = pl.ds(s * ROWS_ACC, ROWS_ACC)
  pltpu.sync_copy(zeros_hbm.at[ra], acc.at[ra])
  pltpu.sync_copy(ones_hbm, ones_v)
  plsc.subcore_barrier()
  base = wid * DEG_BLOCKS * DEG_BLK

  @pl.loop(0, DEG_BLOCKS)
  def _(b):
    pltpu.sync_copy(edges.at[1, pl.ds(base + b * DEG_BLK, DEG_BLK)], idx_v)
    pltpu.sync_copy(ones_v, acc.at[idx_v], add=True)

  plsc.subcore_barrier()
  pltpu.sync_copy(acc.at[ra], out_hbm.at[c, ra])


_degree = pl.kernel(
    _degree_body,
    out_type=jax.ShapeDtypeStruct((N_CORES, N_ACC, 16), jnp.float32),
    mesh=_MESH,
    scratch_types=[
        pltpu.VMEM((DEG_BLK,), jnp.int32),
        pltpu.VMEM((DEG_BLK, 16), jnp.float32),
        pltpu.VMEM_SHARED((N_ACC, 16), jnp.float32),
    ],
    compiler_params=_SC_PARAMS,
)


def _make_conv(feats, blk, n_passes):
  fh = feats // (2 * n_passes)
  n_blocks = E_EDGES // (N_SUBCORES * blk)

  def body(edges, *refs):
    tabs = refs[:2 * n_passes]
    (out_hbm, src_v0, dst_v0, src_v1, dst_v1, rows0, rows1, tab_sp, acc,
     sg0, sg1) = refs[2 * n_passes:]
    c = lax.axis_index("c")
    s = lax.axis_index("s")
    ra = pl.ds(s * ROWS_ACC, ROWS_ACC)
    base = s * n_blocks * blk

    def load_idx(b, sv, dv):
      pltpu.sync_copy(edges.at[0, pl.ds(base + b * blk, blk)], sv)
      pltpu.sync_copy(edges.at[1, pl.ds(base + b * blk, blk)], dv)

    for p in range(n_passes):
      @pl.when(c == 0)
      def _():
        pltpu.sync_copy(tabs[p].at[ra], tab_sp.at[ra])
        pltpu.sync_copy(tabs[p].at[ra], acc.at[ra])

      @pl.when(c != 0)
      def _():
        pltpu.sync_copy(tabs[n_passes + p].at[ra], tab_sp.at[ra])
        pltpu.sync_copy(tabs[n_passes + p].at[ra], acc.at[ra])

      plsc.subcore_barrier()

      load_idx(0, src_v0, dst_v0)
      pltpu.async_copy(tab_sp.at[src_v0], rows0, sg0)
      nh = n_blocks // 2

      @pl.loop(0, nh)
      def _(g):
        load_idx(2 * g + 1, src_v1, dst_v1)
        pltpu.make_async_copy(tab_sp.at[src_v0], rows0, sg0).wait()
        pltpu.async_copy(tab_sp.at[src_v1], rows1, sg1)
        pltpu.sync_copy(rows0, acc.at[dst_v0], add=True)

        @pl.when(g < nh - 1)
        def _():
          load_idx(2 * g + 2, src_v0, dst_v0)
          pltpu.async_copy(tab_sp.at[src_v0], rows0, sg0)

        pltpu.make_async_copy(tab_sp.at[src_v1], rows1, sg1).wait()
        pltpu.sync_copy(rows1, acc.at[dst_v1], add=True)

      plsc.subcore_barrier()
      pltpu.sync_copy(acc.at[ra], out_hbm.at[c * n_passes + p, ra])

  return pl.kernel(
      body,
      out_type=jax.ShapeDtypeStruct((2 * n_passes, N_ACC, fh), jnp.float32),
      mesh=_MESH,
      scratch_types=[
          pltpu.VMEM((blk,), jnp.int32),
          pltpu.VMEM((blk,), jnp.int32),
          pltpu.VMEM((blk,), jnp.int32),
          pltpu.VMEM((blk,), jnp.int32),
          pltpu.VMEM((blk, fh), jnp.float32),
          pltpu.VMEM((blk, fh), jnp.float32),
          pltpu.VMEM_SHARED((N_ACC, fh), jnp.float32),
          pltpu.VMEM_SHARED((N_ACC, fh), jnp.float32),
          pltpu.SemaphoreType.DMA,
          pltpu.SemaphoreType.DMA,
      ],
      compiler_params=_SC_PARAMS,
  )


_conv_h = _make_conv(H_FEATS, 1000, 2)
_conv_out = _make_conv(OUT_FEATS, 1000, 1)


def _tc1_body(x_ref, wg_ref, we_ref, be_ref, h0_ref, h1_ref,
              ta_ref, tb_ref, tc_ref, td_ref, dis_ref):
  xb = x_ref[...]
  logits = jnp.dot(xb, wg_ref[...], preferred_element_type=jnp.float32)
  m = jnp.max(logits, axis=1, keepdims=True)
  l0 = logits[:, 0:1]
  l1 = logits[:, 1:2]
  l2 = logits[:, 2:3]
  g0 = l0 >= m
  g1 = (l1 >= m) & ~g0
  g2 = (l2 >= m) & ~g0 & ~g1
  g3 = ~g0 & ~g1 & ~g2
  h = jnp.zeros((xb.shape[0], H_FEATS), jnp.float32)
  for k, gk in enumerate((g0, g1, g2, g3)):
    hk = jnp.dot(xb, we_ref[k], preferred_element_type=jnp.float32)
    hk = hk + be_ref[k:k + 1, :]
    h = h + gk.astype(jnp.float32) * hk
  deg = h0_ref[0, :, 0:1] + h1_ref[0, :, 0:1] + 1.0
  dis = lax.rsqrt(deg)
  g1s = h * dis
  q = H_FEATS // 4
  ta_ref[...] = g1s[:, 0 * q:1 * q]
  tb_ref[...] = g1s[:, 1 * q:2 * q]
  tc_ref[...] = g1s[:, 2 * q:3 * q]
  td_ref[...] = g1s[:, 3 * q:4 * q]
  dis_ref[...] = dis


def _tc1(x, w_gate, W_experts, b_experts, hist):
  grid = (N_NODES // BN,)
  return pl.pallas_call(
      _tc1_body,
      grid=grid,
      in_specs=[
          pl.BlockSpec((BN, IN_FEATS), lambda i: (i, 0)),
          pl.BlockSpec((IN_FEATS, 4), lambda i: (0, 0)),
          pl.BlockSpec((4, IN_FEATS, H_FEATS), lambda i: (0, 0, 0)),
          pl.BlockSpec((4, H_FEATS), lambda i: (0, 0)),
          pl.BlockSpec((1, BN, 16), lambda i: (0, i, 0)),
          pl.BlockSpec((1, BN, 16), lambda i: (1, i, 0)),
      ],
      out_specs=[
          pl.BlockSpec((BN, H_FEATS // 4), lambda i: (i, 0)),
          pl.BlockSpec((BN, H_FEATS // 4), lambda i: (i, 0)),
          pl.BlockSpec((BN, H_FEATS // 4), lambda i: (i, 0)),
          pl.BlockSpec((BN, H_FEATS // 4), lambda i: (i, 0)),
          pl.BlockSpec((BN, 1), lambda i: (i, 0)),
      ],
      out_shape=[
          jax.ShapeDtypeStruct((N_ACC, H_FEATS // 4), jnp.float32),
          jax.ShapeDtypeStruct((N_ACC, H_FEATS // 4), jnp.float32),
          jax.ShapeDtypeStruct((N_ACC, H_FEATS // 4), jnp.float32),
          jax.ShapeDtypeStruct((N_ACC, H_FEATS // 4), jnp.float32),
          jax.ShapeDtypeStruct((N_NODES, 1), jnp.float32),
      ],
  )(x, w_gate, W_experts, b_experts, hist, hist)


def _tc2_body(p0_ref, p1_ref, p2_ref, p3_ref, dis_ref, b1_ref, w2_ref,
              ta_ref, tb_ref):
  dis = dis_ref[...]
  h = jnp.concatenate([p0_ref[0], p1_ref[0], p2_ref[0], p3_ref[0]], axis=1)
  h = h * dis + b1_ref[...]
  h = jnp.maximum(h, 0.0)
  g2 = jnp.dot(h, w2_ref[...], preferred_element_type=jnp.float32) * dis
  ta_ref[...] = g2[:, :OUT_FEATS // 2]
  tb_ref[...] = g2[:, OUT_FEATS // 2:]


def _tc2(p, dis, b1, w2):
  grid = (N_NODES // BN,)
  qspec = lambda j: pl.BlockSpec((1, BN, H_FEATS // 4),
                                 lambda i, j=j: (j, i, 0))
  return pl.pallas_call(
      _tc2_body,
      grid=grid,
      in_specs=[
          qspec(0), qspec(1), qspec(2), qspec(3),
          pl.BlockSpec((BN, 1), lambda i: (i, 0)),
          pl.BlockSpec((1, H_FEATS), lambda i: (0, 0)),
          pl.BlockSpec((H_FEATS, OUT_FEATS), lambda i: (0, 0)),
      ],
      out_specs=[
          pl.BlockSpec((BN, OUT_FEATS // 2), lambda i: (i, 0)),
          pl.BlockSpec((BN, OUT_FEATS // 2), lambda i: (i, 0)),
      ],
      out_shape=[
          jax.ShapeDtypeStruct((N_ACC, OUT_FEATS // 2), jnp.float32),
          jax.ShapeDtypeStruct((N_ACC, OUT_FEATS // 2), jnp.float32),
      ],
  )(p, p, p, p, dis, b1, w2)


def _tc3_body(q0_ref, q1_ref, dis_ref, b2_ref, out_ref):
  z = jnp.concatenate([q0_ref[0], q1_ref[0]], axis=1)
  z = z * dis_ref[...] + b2_ref[...]
  m = jnp.max(z, axis=1, keepdims=True)
  zs = z - m
  out_ref[...] = zs - jnp.log(jnp.sum(jnp.exp(zs), axis=1, keepdims=True))


def _tc3(q, dis, b2):
  grid = (N_NODES // BN,)
  return pl.pallas_call(
      _tc3_body,
      grid=grid,
      in_specs=[
          pl.BlockSpec((1, BN, OUT_FEATS // 2), lambda i: (0, i, 0)),
          pl.BlockSpec((1, BN, OUT_FEATS // 2), lambda i: (1, i, 0)),
          pl.BlockSpec((BN, 1), lambda i: (i, 0)),
          pl.BlockSpec((1, OUT_FEATS), lambda i: (0, 0)),
      ],
      out_specs=pl.BlockSpec((BN, OUT_FEATS), lambda i: (i, 0)),
      out_shape=jax.ShapeDtypeStruct((N_NODES, OUT_FEATS), jnp.float32),
  )(q, q, dis, b2)


@jax.jit
def kernel(x, edge_index, w_gate, W_experts, b_experts, b1, W2, b2):
  hist = _degree(edge_index, jnp.ones((DEG_BLK, 16), jnp.float32),
                 jnp.zeros((N_ACC, 16), jnp.float32))
  ta, tb, tc, td, dis = _tc1(x, w_gate, W_experts, b_experts, hist)
  p = _conv_h(edge_index, ta, tb, tc, td)
  t2a, t2b = _tc2(p, dis, b1.reshape(1, -1), W2)
  q = _conv_out(edge_index, t2a, t2b)
  return _tc3(q, dis, b2.reshape(1, -1))


# speedup vs baseline: 24.2455x; 1.0159x over previous
import jax
import jax.numpy as jnp
from jax import lax
from jax.experimental import pallas as pl
from jax.experimental.pallas import tpu as pltpu
from jax.experimental.pallas import tpu_sc as plsc

N_NODES = 10000
E_EDGES = 320000
IN_FEATS = 128
H_FEATS = 128
OUT_FEATS = 64

N_CORES = 2
N_SUBCORES = 16
N_ACC = 10112
ROWS_ACC = N_ACC // N_SUBCORES
BN = 1000

DEG_BLK = 1000
DEG_BLOCKS = E_EDGES // (N_CORES * N_SUBCORES * DEG_BLK)

_MESH = plsc.VectorSubcoreMesh(core_axis_name="c", subcore_axis_name="s")
_SC_PARAMS = pltpu.CompilerParams(use_tc_tiling_on_sc=False)


def _degree_body(edges, ones_hbm, zeros_hbm, out_hbm, idx_v, ones_v, acc):
  c = lax.axis_index("c")
  s = lax.axis_index("s")
  wid = c * N_SUBCORES + s
  ra = pl.ds(s * ROWS_ACC, ROWS_ACC)
  pltpu.sync_copy(zeros_hbm.at[ra], acc.at[ra])
  pltpu.sync_copy(ones_hbm, ones_v)
  plsc.subcore_barrier()
  base = wid * DEG_BLOCKS * DEG_BLK

  @pl.loop(0, DEG_BLOCKS)
  def _(b):
    pltpu.sync_copy(edges.at[1, pl.ds(base + b * DEG_BLK, DEG_BLK)], idx_v)
    pltpu.sync_copy(ones_v, acc.at[idx_v], add=True)

  plsc.subcore_barrier()
  pltpu.sync_copy(acc.at[ra], out_hbm.at[c, ra])


_degree = pl.kernel(
    _degree_body,
    out_type=jax.ShapeDtypeStruct((N_CORES, N_ACC, 16), jnp.float32),
    mesh=_MESH,
    scratch_types=[
        pltpu.VMEM((DEG_BLK,), jnp.int32),
        pltpu.VMEM((DEG_BLK, 16), jnp.float32),
        pltpu.VMEM_SHARED((N_ACC, 16), jnp.float32),
    ],
    compiler_params=_SC_PARAMS,
)


def _make_conv(feats, blk, n_passes):
  fh = feats // (2 * n_passes)
  n_blocks = E_EDGES // (N_SUBCORES * blk)

  def body(edges, *refs):
    tabs = refs[:2 * n_passes]
    (out_hbm, src_v0, dst_v0, src_v1, dst_v1, rows0, rows1, tab_sp, acc,
     sg0, sg1) = refs[2 * n_passes:]
    c = lax.axis_index("c")
    s = lax.axis_index("s")
    ra = pl.ds(s * ROWS_ACC, ROWS_ACC)
    base = s * n_blocks * blk

    def load_idx(b, sv, dv):
      pltpu.sync_copy(edges.at[0, pl.ds(base + b * blk, blk)], sv)
      pltpu.sync_copy(edges.at[1, pl.ds(base + b * blk, blk)], dv)

    for p in range(n_passes):
      @pl.when(c == 0)
      def _():
        pltpu.sync_copy(tabs[p].at[ra], tab_sp.at[ra])
        pltpu.sync_copy(tabs[p].at[ra], acc.at[ra])

      @pl.when(c != 0)
      def _():
        pltpu.sync_copy(tabs[n_passes + p].at[ra], tab_sp.at[ra])
        pltpu.sync_copy(tabs[n_passes + p].at[ra], acc.at[ra])

      plsc.subcore_barrier()

      load_idx(0, src_v0, dst_v0)
      pltpu.async_copy(tab_sp.at[src_v0], rows0, sg0)
      nh = n_blocks // 2

      @pl.loop(0, nh)
      def _(g):
        load_idx(2 * g + 1, src_v1, dst_v1)
        pltpu.make_async_copy(tab_sp.at[src_v0], rows0, sg0).wait()
        pltpu.async_copy(tab_sp.at[src_v1], rows1, sg1)
        pltpu.sync_copy(rows0, acc.at[dst_v0], add=True)

        @pl.when(g < nh - 1)
        def _():
          load_idx(2 * g + 2, src_v0, dst_v0)
          pltpu.async_copy(tab_sp.at[src_v0], rows0, sg0)

        pltpu.make_async_copy(tab_sp.at[src_v1], rows1, sg1).wait()
        pltpu.sync_copy(rows1, acc.at[dst_v1], add=True)

      plsc.subcore_barrier()
      pltpu.sync_copy(acc.at[ra], out_hbm.at[c * n_passes + p, ra])

  return pl.kernel(
      body,
      out_type=jax.ShapeDtypeStruct((2 * n_passes, N_ACC, fh), jnp.float32),
      mesh=_MESH,
      scratch_types=[
          pltpu.VMEM((blk,), jnp.int32),
          pltpu.VMEM((blk,), jnp.int32),
          pltpu.VMEM((blk,), jnp.int32),
          pltpu.VMEM((blk,), jnp.int32),
          pltpu.VMEM((blk, fh), jnp.float32),
          pltpu.VMEM((blk, fh), jnp.float32),
          pltpu.VMEM_SHARED((N_ACC, fh), jnp.float32),
          pltpu.VMEM_SHARED((N_ACC, fh), jnp.float32),
          pltpu.SemaphoreType.DMA,
          pltpu.SemaphoreType.DMA,
      ],
      compiler_params=_SC_PARAMS,
  )


_conv_h = _make_conv(H_FEATS, 1000, 2)
_conv_out = _make_conv(OUT_FEATS, 1000, 1)


def _tc1a_body(x_ref, wg_ref, we_ref, be_ref, h_ref):
  xb = x_ref[...]
  logits = jnp.dot(xb, wg_ref[...], preferred_element_type=jnp.float32)
  m = jnp.max(logits, axis=1, keepdims=True)
  l0 = logits[:, 0:1]
  l1 = logits[:, 1:2]
  l2 = logits[:, 2:3]
  g0 = l0 >= m
  g1 = (l1 >= m) & ~g0
  g2 = (l2 >= m) & ~g0 & ~g1
  g3 = ~g0 & ~g1 & ~g2
  h = jnp.zeros((xb.shape[0], H_FEATS), jnp.float32)
  for k, gk in enumerate((g0, g1, g2, g3)):
    hk = jnp.dot(xb, we_ref[k], preferred_element_type=jnp.float32)
    hk = hk + be_ref[k:k + 1, :]
    h = h + gk.astype(jnp.float32) * hk
  h_ref[...] = h


def _tc1a(x, w_gate, W_experts, b_experts):
  grid = (N_NODES // BN,)
  return pl.pallas_call(
      _tc1a_body,
      grid=grid,
      in_specs=[
          pl.BlockSpec((BN, IN_FEATS), lambda i: (i, 0)),
          pl.BlockSpec((IN_FEATS, 4), lambda i: (0, 0)),
          pl.BlockSpec((4, IN_FEATS, H_FEATS), lambda i: (0, 0, 0)),
          pl.BlockSpec((4, H_FEATS), lambda i: (0, 0)),
      ],
      out_specs=pl.BlockSpec((BN, H_FEATS), lambda i: (i, 0)),
      out_shape=jax.ShapeDtypeStruct((N_NODES, H_FEATS), jnp.float32),
  )(x, w_gate, W_experts, b_experts)


def _tc1b_body(h_ref, h0_ref, h1_ref, ta_ref, tb_ref, tc_ref, td_ref,
               dis_ref):
  deg = h0_ref[0, :, 0:1] + h1_ref[0, :, 0:1] + 1.0
  dis = lax.rsqrt(deg)
  g1s = h_ref[...] * dis
  q = H_FEATS // 4
  ta_ref[...] = g1s[:, 0 * q:1 * q]
  tb_ref[...] = g1s[:, 1 * q:2 * q]
  tc_ref[...] = g1s[:, 2 * q:3 * q]
  td_ref[...] = g1s[:, 3 * q:4 * q]
  dis_ref[...] = dis


def _tc1b(h, hist):
  grid = (N_NODES // BN,)
  return pl.pallas_call(
      _tc1b_body,
      grid=grid,
      in_specs=[
          pl.BlockSpec((BN, H_FEATS), lambda i: (i, 0)),
          pl.BlockSpec((1, BN, 16), lambda i: (0, i, 0)),
          pl.BlockSpec((1, BN, 16), lambda i: (1, i, 0)),
      ],
      out_specs=[
          pl.BlockSpec((BN, H_FEATS // 4), lambda i: (i, 0)),
          pl.BlockSpec((BN, H_FEATS // 4), lambda i: (i, 0)),
          pl.BlockSpec((BN, H_FEATS // 4), lambda i: (i, 0)),
          pl.BlockSpec((BN, H_FEATS // 4), lambda i: (i, 0)),
          pl.BlockSpec((BN, 1), lambda i: (i, 0)),
      ],
      out_shape=[
          jax.ShapeDtypeStruct((N_ACC, H_FEATS // 4), jnp.float32),
          jax.ShapeDtypeStruct((N_ACC, H_FEATS // 4), jnp.float32),
          jax.ShapeDtypeStruct((N_ACC, H_FEATS // 4), jnp.float32),
          jax.ShapeDtypeStruct((N_ACC, H_FEATS // 4), jnp.float32),
          jax.ShapeDtypeStruct((N_NODES, 1), jnp.float32),
      ],
  )(h, hist, hist)


def _tc2_body(p0_ref, p1_ref, p2_ref, p3_ref, dis_ref, b1_ref, w2_ref,
              ta_ref, tb_ref):
  dis = dis_ref[...]
  h = jnp.concatenate([p0_ref[0], p1_ref[0], p2_ref[0], p3_ref[0]], axis=1)
  h = h * dis + b1_ref[...]
  h = jnp.maximum(h, 0.0)
  g2 = jnp.dot(h, w2_ref[...], preferred_element_type=jnp.float32) * dis
  ta_ref[...] = g2[:, :OUT_FEATS // 2]
  tb_ref[...] = g2[:, OUT_FEATS // 2:]


def _tc2(p, dis, b1, w2):
  grid = (N_NODES // BN,)
  qspec = lambda j: pl.BlockSpec((1, BN, H_FEATS // 4),
                                 lambda i, j=j: (j, i, 0))
  return pl.pallas_call(
      _tc2_body,
      grid=grid,
      in_specs=[
          qspec(0), qspec(1), qspec(2), qspec(3),
          pl.BlockSpec((BN, 1), lambda i: (i, 0)),
          pl.BlockSpec((1, H_FEATS), lambda i: (0, 0)),
          pl.BlockSpec((H_FEATS, OUT_FEATS), lambda i: (0, 0)),
      ],
      out_specs=[
          pl.BlockSpec((BN, OUT_FEATS // 2), lambda i: (i, 0)),
          pl.BlockSpec((BN, OUT_FEATS // 2), lambda i: (i, 0)),
      ],
      out_shape=[
          jax.ShapeDtypeStruct((N_ACC, OUT_FEATS // 2), jnp.float32),
          jax.ShapeDtypeStruct((N_ACC, OUT_FEATS // 2), jnp.float32),
      ],
  )(p, p, p, p, dis, b1, w2)


def _tc3_body(q0_ref, q1_ref, dis_ref, b2_ref, out_ref):
  z = jnp.concatenate([q0_ref[0], q1_ref[0]], axis=1)
  z = z * dis_ref[...] + b2_ref[...]
  m = jnp.max(z, axis=1, keepdims=True)
  zs = z - m
  out_ref[...] = zs - jnp.log(jnp.sum(jnp.exp(zs), axis=1, keepdims=True))


def _tc3(q, dis, b2):
  grid = (N_NODES // BN,)
  return pl.pallas_call(
      _tc3_body,
      grid=grid,
      in_specs=[
          pl.BlockSpec((1, BN, OUT_FEATS // 2), lambda i: (0, i, 0)),
          pl.BlockSpec((1, BN, OUT_FEATS // 2), lambda i: (1, i, 0)),
          pl.BlockSpec((BN, 1), lambda i: (i, 0)),
          pl.BlockSpec((1, OUT_FEATS), lambda i: (0, 0)),
      ],
      out_specs=pl.BlockSpec((BN, OUT_FEATS), lambda i: (i, 0)),
      out_shape=jax.ShapeDtypeStruct((N_NODES, OUT_FEATS), jnp.float32),
  )(q, q, dis, b2)


@jax.jit
def kernel(x, edge_index, w_gate, W_experts, b_experts, b1, W2, b2):
  hist = _degree(edge_index, jnp.ones((DEG_BLK, 16), jnp.float32),
                 jnp.zeros((N_ACC, 16), jnp.float32))
  h = _tc1a(x, w_gate, W_experts, b_experts)
  ta, tb, tc, td, dis = _tc1b(h, hist)
  p = _conv_h(edge_index, ta, tb, tc, td)
  t2a, t2b = _tc2(p, dis, b1.reshape(1, -1), W2)
  q = _conv_out(edge_index, t2a, t2b)
  return _tc3(q, dis, b2.reshape(1, -1))


# speedup vs baseline: 24.3711x; 1.0052x over previous
import jax
import jax.numpy as jnp
from jax import lax
from jax.experimental import pallas as pl
from jax.experimental.pallas import tpu as pltpu
from jax.experimental.pallas import tpu_sc as plsc

N_NODES = 10000
E_EDGES = 320000
IN_FEATS = 128
H_FEATS = 128
OUT_FEATS = 64

N_CORES = 2
N_SUBCORES = 16
N_ACC = 10112
ROWS_ACC = N_ACC // N_SUBCORES
BN = 1000

DEG_BLK = 2000
DEG_BLOCKS = E_EDGES // (N_CORES * N_SUBCORES * DEG_BLK)

_MESH = plsc.VectorSubcoreMesh(core_axis_name="c", subcore_axis_name="s")
_SC_PARAMS = pltpu.CompilerParams(use_tc_tiling_on_sc=False)


def _degree_body(edges, ones_hbm, zeros_hbm, out_hbm, idx_v, ones_v, acc):
  c = lax.axis_index("c")
  s = lax.axis_index("s")
  wid = c * N_SUBCORES + s
  ra = pl.ds(s * ROWS_ACC, ROWS_ACC)
  pltpu.sync_copy(zeros_hbm.at[ra], acc.at[ra])
  pltpu.sync_copy(ones_hbm, ones_v)
  plsc.subcore_barrier()
  base = wid * DEG_BLOCKS * DEG_BLK

  @pl.loop(0, DEG_BLOCKS)
  def _(b):
    pltpu.sync_copy(edges.at[1, pl.ds(base + b * DEG_BLK, DEG_BLK)], idx_v)
    pltpu.sync_copy(ones_v, acc.at[idx_v], add=True)

  plsc.subcore_barrier()
  pltpu.sync_copy(acc.at[ra], out_hbm.at[c, ra])


_degree = pl.kernel(
    _degree_body,
    out_type=jax.ShapeDtypeStruct((N_CORES, N_ACC, 16), jnp.float32),
    mesh=_MESH,
    scratch_types=[
        pltpu.VMEM((DEG_BLK,), jnp.int32),
        pltpu.VMEM((DEG_BLK, 16), jnp.float32),
        pltpu.VMEM_SHARED((N_ACC, 16), jnp.float32),
    ],
    compiler_params=_SC_PARAMS,
)


def _make_conv(feats, blk, n_passes):
  fh = feats // (2 * n_passes)
  n_blocks = E_EDGES // (N_SUBCORES * blk)

  def body(edges, *refs):
    tabs = refs[:2 * n_passes]
    (out_hbm, src_v0, dst_v0, src_v1, dst_v1, rows0, rows1, tab_sp, acc,
     sg0, sg1) = refs[2 * n_passes:]
    c = lax.axis_index("c")
    s = lax.axis_index("s")
    ra = pl.ds(s * ROWS_ACC, ROWS_ACC)
    base = s * n_blocks * blk

    def load_idx(b, sv, dv):
      pltpu.sync_copy(edges.at[0, pl.ds(base + b * blk, blk)], sv)
      pltpu.sync_copy(edges.at[1, pl.ds(base + b * blk, blk)], dv)

    for p in range(n_passes):
      @pl.when(c == 0)
      def _():
        pltpu.sync_copy(tabs[p].at[ra], tab_sp.at[ra])
        pltpu.sync_copy(tabs[p].at[ra], acc.at[ra])

      @pl.when(c != 0)
      def _():
        pltpu.sync_copy(tabs[n_passes + p].at[ra], tab_sp.at[ra])
        pltpu.sync_copy(tabs[n_passes + p].at[ra], acc.at[ra])

      plsc.subcore_barrier()

      load_idx(0, src_v0, dst_v0)
      pltpu.async_copy(tab_sp.at[src_v0], rows0, sg0)
      nh = n_blocks // 2

      @pl.loop(0, nh)
      def _(g):
        load_idx(2 * g + 1, src_v1, dst_v1)
        pltpu.make_async_copy(tab_sp.at[src_v0], rows0, sg0).wait()
        pltpu.async_copy(tab_sp.at[src_v1], rows1, sg1)
        pltpu.sync_copy(rows0, acc.at[dst_v0], add=True)

        @pl.when(g < nh - 1)
        def _():
          load_idx(2 * g + 2, src_v0, dst_v0)
          pltpu.async_copy(tab_sp.at[src_v0], rows0, sg0)

        pltpu.make_async_copy(tab_sp.at[src_v1], rows1, sg1).wait()
        pltpu.sync_copy(rows1, acc.at[dst_v1], add=True)

      plsc.subcore_barrier()
      pltpu.sync_copy(acc.at[ra], out_hbm.at[c * n_passes + p, ra])

  return pl.kernel(
      body,
      out_type=jax.ShapeDtypeStruct((2 * n_passes, N_ACC, fh), jnp.float32),
      mesh=_MESH,
      scratch_types=[
          pltpu.VMEM((blk,), jnp.int32),
          pltpu.VMEM((blk,), jnp.int32),
          pltpu.VMEM((blk,), jnp.int32),
          pltpu.VMEM((blk,), jnp.int32),
          pltpu.VMEM((blk, fh), jnp.float32),
          pltpu.VMEM((blk, fh), jnp.float32),
          pltpu.VMEM_SHARED((N_ACC, fh), jnp.float32),
          pltpu.VMEM_SHARED((N_ACC, fh), jnp.float32),
          pltpu.SemaphoreType.DMA,
          pltpu.SemaphoreType.DMA,
      ],
      compiler_params=_SC_PARAMS,
  )


_conv_h = _make_conv(H_FEATS, 1000, 2)
_conv_out = _make_conv(OUT_FEATS, 1000, 1)


def _tc1a_body(x_ref, wg_ref, we_ref, be_ref, h_ref):
  xb = x_ref[...]
  logits = jnp.dot(xb, wg_ref[...], preferred_element_type=jnp.float32)
  m = jnp.max(logits, axis=1, keepdims=True)
  l0 = logits[:, 0:1]
  l1 = logits[:, 1:2]
  l2 = logits[:, 2:3]
  g0 = l0 >= m
  g1 = (l1 >= m) & ~g0
  g2 = (l2 >= m) & ~g0 & ~g1
  g3 = ~g0 & ~g1 & ~g2
  h = jnp.zeros((xb.shape[0], H_FEATS), jnp.float32)
  for k, gk in enumerate((g0, g1, g2, g3)):
    hk = jnp.dot(xb, we_ref[k], preferred_element_type=jnp.float32)
    hk = hk + be_ref[k:k + 1, :]
    h = h + gk.astype(jnp.float32) * hk
  h_ref[...] = h


def _tc1a(x, w_gate, W_experts, b_experts):
  grid = (N_NODES // BN,)
  return pl.pallas_call(
      _tc1a_body,
      grid=grid,
      in_specs=[
          pl.BlockSpec((BN, IN_FEATS), lambda i: (i, 0)),
          pl.BlockSpec((IN_FEATS, 4), lambda i: (0, 0)),
          pl.BlockSpec((4, IN_FEATS, H_FEATS), lambda i: (0, 0, 0)),
          pl.BlockSpec((4, H_FEATS), lambda i: (0, 0)),
      ],
      out_specs=pl.BlockSpec((BN, H_FEATS), lambda i: (i, 0)),
      out_shape=jax.ShapeDtypeStruct((N_NODES, H_FEATS), jnp.float32),
  )(x, w_gate, W_experts, b_experts)


def _tc1b_body(h_ref, h0_ref, h1_ref, ta_ref, tb_ref, tc_ref, td_ref,
               dis_ref):
  deg = h0_ref[0, :, 0:1] + h1_ref[0, :, 0:1] + 1.0
  dis = lax.rsqrt(deg)
  g1s = h_ref[...] * dis
  q = H_FEATS // 4
  ta_ref[...] = g1s[:, 0 * q:1 * q]
  tb_ref[...] = g1s[:, 1 * q:2 * q]
  tc_ref[...] = g1s[:, 2 * q:3 * q]
  td_ref[...] = g1s[:, 3 * q:4 * q]
  dis_ref[...] = dis


def _tc1b(h, hist):
  grid = (N_NODES // BN,)
  return pl.pallas_call(
      _tc1b_body,
      grid=grid,
      in_specs=[
          pl.BlockSpec((BN, H_FEATS), lambda i: (i, 0)),
          pl.BlockSpec((1, BN, 16), lambda i: (0, i, 0)),
          pl.BlockSpec((1, BN, 16), lambda i: (1, i, 0)),
      ],
      out_specs=[
          pl.BlockSpec((BN, H_FEATS // 4), lambda i: (i, 0)),
          pl.BlockSpec((BN, H_FEATS // 4), lambda i: (i, 0)),
          pl.BlockSpec((BN, H_FEATS // 4), lambda i: (i, 0)),
          pl.BlockSpec((BN, H_FEATS // 4), lambda i: (i, 0)),
          pl.BlockSpec((BN, 1), lambda i: (i, 0)),
      ],
      out_shape=[
          jax.ShapeDtypeStruct((N_ACC, H_FEATS // 4), jnp.float32),
          jax.ShapeDtypeStruct((N_ACC, H_FEATS // 4), jnp.float32),
          jax.ShapeDtypeStruct((N_ACC, H_FEATS // 4), jnp.float32),
          jax.ShapeDtypeStruct((N_ACC, H_FEATS // 4), jnp.float32),
          jax.ShapeDtypeStruct((N_NODES, 1), jnp.float32),
      ],
  )(h, hist, hist)


def _tc2_body(p0_ref, p1_ref, p2_ref, p3_ref, dis_ref, b1_ref, w2_ref,
              ta_ref, tb_ref):
  dis = dis_ref[...]
  h = jnp.concatenate([p0_ref[0], p1_ref[0], p2_ref[0], p3_ref[0]], axis=1)
  h = h * dis + b1_ref[...]
  h = jnp.maximum(h, 0.0)
  g2 = jnp.dot(h, w2_ref[...], preferred_element_type=jnp.float32) * dis
  ta_ref[...] = g2[:, :OUT_FEATS // 2]
  tb_ref[...] = g2[:, OUT_FEATS // 2:]


def _tc2(p, dis, b1, w2):
  grid = (N_NODES // BN,)
  qspec = lambda j: pl.BlockSpec((1, BN, H_FEATS // 4),
                                 lambda i, j=j: (j, i, 0))
  return pl.pallas_call(
      _tc2_body,
      grid=grid,
      in_specs=[
          qspec(0), qspec(1), qspec(2), qspec(3),
          pl.BlockSpec((BN, 1), lambda i: (i, 0)),
          pl.BlockSpec((1, H_FEATS), lambda i: (0, 0)),
          pl.BlockSpec((H_FEATS, OUT_FEATS), lambda i: (0, 0)),
      ],
      out_specs=[
          pl.BlockSpec((BN, OUT_FEATS // 2), lambda i: (i, 0)),
          pl.BlockSpec((BN, OUT_FEATS // 2), lambda i: (i, 0)),
      ],
      out_shape=[
          jax.ShapeDtypeStruct((N_ACC, OUT_FEATS // 2), jnp.float32),
          jax.ShapeDtypeStruct((N_ACC, OUT_FEATS // 2), jnp.float32),
      ],
  )(p, p, p, p, dis, b1, w2)


def _tc3_body(q0_ref, q1_ref, dis_ref, b2_ref, out_ref):
  z = jnp.concatenate([q0_ref[0], q1_ref[0]], axis=1)
  z = z * dis_ref[...] + b2_ref[...]
  m = jnp.max(z, axis=1, keepdims=True)
  zs = z - m
  out_ref[...] = zs - jnp.log(jnp.sum(jnp.exp(zs), axis=1, keepdims=True))


def _tc3(q, dis, b2):
  grid = (N_NODES // BN,)
  return pl.pallas_call(
      _tc3_body,
      grid=grid,
      in_specs=[
          pl.BlockSpec((1, BN, OUT_FEATS // 2), lambda i: (0, i, 0)),
          pl.BlockSpec((1, BN, OUT_FEATS // 2), lambda i: (1, i, 0)),
          pl.BlockSpec((BN, 1), lambda i: (i, 0)),
          pl.BlockSpec((1, OUT_FEATS), lambda i: (0, 0)),
      ],
      out_specs=pl.BlockSpec((BN, OUT_FEATS), lambda i: (i, 0)),
      out_shape=jax.ShapeDtypeStruct((N_NODES, OUT_FEATS), jnp.float32),
  )(q, q, dis, b2)


@jax.jit
def kernel(x, edge_index, w_gate, W_experts, b_experts, b1, W2, b2):
  hist = _degree(edge_index, jnp.ones((DEG_BLK, 16), jnp.float32),
                 jnp.zeros((N_ACC, 16), jnp.float32))
  h = _tc1a(x, w_gate, W_experts, b_experts)
  ta, tb, tc, td, dis = _tc1b(h, hist)
  p = _conv_h(edge_index, ta, tb, tc, td)
  t2a, t2b = _tc2(p, dis, b1.reshape(1, -1), W2)
  q = _conv_out(edge_index, t2a, t2b)
  return _tc3(q, dis, b2.reshape(1, -1))
